# Initial kernel scaffold; baseline (speedup 1.0000x reference)
import jax
import jax.numpy as jnp
from jax.experimental import pallas as pl


def kernel(node_feats, edge_feats, edge_index, gc_pn_W, gc_pn_b, gc_pe1_W, gc_pe1_b, gc_pe2_W, gc_pe2_b, gc_et_W, gc_et_b, gc_gru_Wih, gc_gru_Whh, gc_gru_bih, gc_gru_bhh, l0_pe_W, l0_pe_b, l0_pn_W, l0_pn_b, l0_gru_Wih, l0_gru_Whh, l0_gru_bih, l0_gru_bhh, l1_pe_W, l1_pe_b, l1_pn_W, l1_pn_b, l1_gru_Wih, l1_gru_Whh, l1_gru_bih, l1_gru_bhh, pred_W, pred_b):
    raise NotImplementedError("write your pallas kernel here")


# speedup vs baseline: 4.2994x; 4.2994x over previous
import functools

import jax
import jax.numpy as jnp
from jax import lax
from jax.experimental import pallas as pl
from jax.experimental.pallas import tpu as pltpu
from jax.experimental.pallas import tpu_sc as plsc

V = 10000
E = 320000
D = 128
DE = 16
G = 128

NC = 2
NS = 16
NW = NC * NS
V_PAD = 10240
E_PAD = 327680
EPW = E_PAD // NW
CH = 128
NCHUNK = EPW // CH
RPT = V_PAD // NS
DUMMY = V_PAD - 1

_mesh = plsc.VectorSubcoreMesh(core_axis_name="c", subcore_axis_name="s")


def _leaky_v(x):
    return jnp.where(x > 0, x, x * 0.01)


def _wid():
    return lax.axis_index("s") * NC + lax.axis_index("c")


@functools.partial(
    pl.kernel,
    out_type=(
        jax.ShapeDtypeStruct((E_PAD, G), jnp.float32),
        jax.ShapeDtypeStruct((E_PAD,), jnp.float32),
    ),
    mesh=_mesh,
    scratch_types=[
        pltpu.VMEM((CH,), jnp.int32),
        pltpu.VMEM((CH,), jnp.int32),
        pltpu.VMEM((CH, G), jnp.float32),
        pltpu.VMEM((CH, G), jnp.float32),
        pltpu.VMEM((CH,), jnp.float32),
        pltpu.SemaphoreType.DMA,
    ],
)
def _g1_kernel(src_hbm, dst_hbm, hvp1_hbm, sn_hbm, efp_hbm,
               he1_hbm, sdn_hbm,
               idx_s, idx_d, rows, efb, dnv, sem):
    wid = _wid()

    def chunk(ch, _):
        base = wid * EPW + ch * CH
        pltpu.sync_copy(src_hbm.at[pl.ds(base, CH)], idx_s)
        pltpu.sync_copy(dst_hbm.at[pl.ds(base, CH)], idx_d)
        pltpu.sync_copy(efp_hbm.at[pl.ds(base, CH)], efb)
        pltpu.async_copy(hvp1_hbm.at[idx_s], rows, sem).wait()
        pltpu.async_copy(sn_hbm.at[idx_d], dnv, sem).wait()

        def edge(i, _):
            for j in range(G // 16):
                u = rows[i, pl.ds(j * 16, 16)] + efb[i, pl.ds(j * 16, 16)]
                rows[i, pl.ds(j * 16, 16)] = _leaky_v(u)
            return 0

        lax.fori_loop(0, CH, edge, 0)
        pltpu.sync_copy(rows, he1_hbm.at[pl.ds(base, CH)])
        pltpu.sync_copy(dnv, sdn_hbm.at[pl.ds(base, CH)])
        return 0

    lax.fori_loop(0, NCHUNK, chunk, 0)


@functools.partial(
    pl.kernel,
    out_type=(
        jax.ShapeDtypeStruct((E_PAD,), jnp.float32),
        jax.ShapeDtypeStruct((E_PAD,), jnp.float32),
    ),
    mesh=_mesh,
    scratch_types=[
        pltpu.VMEM((CH,), jnp.int32),
        pltpu.VMEM((CH,), jnp.int32),
        pltpu.VMEM((CH,), jnp.float32),
        pltpu.VMEM((CH,), jnp.float32),
        pltpu.SemaphoreType.DMA,
    ],
)
def _gat2_kernel(src_hbm, dst_hbm, dsc_hbm, ssc_hbm,
                 de_hbm, se_hbm,
                 idx_s, idx_d, dval, sval, sem):
    wid = _wid()

    def chunk(ch, _):
        base = wid * EPW + ch * CH
        pltpu.sync_copy(src_hbm.at[pl.ds(base, CH)], idx_s)
        pltpu.sync_copy(dst_hbm.at[pl.ds(base, CH)], idx_d)
        pltpu.async_copy(dsc_hbm.at[idx_d], dval, sem).wait()
        pltpu.async_copy(ssc_hbm.at[idx_s], sval, sem).wait()
        pltpu.sync_copy(dval, de_hbm.at[pl.ds(base, CH)])
        pltpu.sync_copy(sval, se_hbm.at[pl.ds(base, CH)])
        return 0

    lax.fori_loop(0, NCHUNK, chunk, 0)


@functools.partial(
    pl.kernel,
    out_type=jax.ShapeDtypeStruct((NC, V_PAD, G), jnp.float32),
    mesh=_mesh,
    scratch_types=[
        pltpu.VMEM((CH,), jnp.int32),
        pltpu.VMEM((1, CH), jnp.int32),
        pltpu.VMEM((CH, G), jnp.float32),
        pltpu.VMEM((CH, 16), jnp.float32),
        pltpu.VMEM((16, G), jnp.float32),
        pltpu.VMEM_SHARED((V_PAD, G), jnp.float32),
        pltpu.SemaphoreType.DMA,
    ],
)
def _scat_kernel(rid_hbm, dst_hbm, tab_hbm, e16_hbm,
                 agg_out,
                 idx_s, idx_d, rows, e16b, zrow, agg_acc, sem):
    cid = lax.axis_index("c")
    sid = lax.axis_index("s")
    wid = sid * NC + cid

    zf = jnp.zeros((16,), jnp.float32)
    for i in range(16):
        for j in range(G // 16):
            zrow[i, pl.ds(j * 16, 16)] = zf
    base_r = sid * RPT

    def zb(t, _):
        pltpu.sync_copy(zrow, agg_acc.at[pl.ds(base_r + t * 16, 16)])
        return 0

    lax.fori_loop(0, RPT // 16, zb, 0)
    plsc.subcore_barrier()

    def chunk(ch, _):
        base = wid * EPW + ch * CH
        pltpu.sync_copy(rid_hbm.at[pl.ds(base, CH)], idx_s)
        pltpu.sync_copy(dst_hbm.at[pl.ds(base, CH)], idx_d.at[0])
        pltpu.sync_copy(e16_hbm.at[pl.ds(base, CH)], e16b)
        pltpu.async_copy(tab_hbm.at[idx_s], rows, sem).wait()

        def edge(i, _):
            eb = e16b[i]
            for j in range(G // 16):
                rows[i, pl.ds(j * 16, 16)] = rows[i, pl.ds(j * 16, 16)] * eb
            return 0

        lax.fori_loop(0, CH, edge, 0)
        pltpu.sync_copy(rows, agg_acc.at[idx_d.at[0]], add=True)
        return 0

    lax.fori_loop(0, NCHUNK, chunk, 0)
    plsc.subcore_barrier()
    pltpu.sync_copy(agg_acc.at[pl.ds(base_r, RPT)],
                    agg_out.at[cid, pl.ds(base_r, RPT)])


@functools.partial(
    pl.kernel,
    out_type=jax.ShapeDtypeStruct((NC, V_PAD, G), jnp.float32),
    mesh=_mesh,
    scratch_types=[
        pltpu.VMEM((1, CH), jnp.int32),
        pltpu.VMEM((CH, 16), jnp.float32),
        pltpu.VMEM((CH, G), jnp.float32),
        pltpu.VMEM((16, G), jnp.float32),
        pltpu.VMEM_SHARED((V_PAD, G), jnp.float32),
        pltpu.SemaphoreType.DMA,
    ],
)
def _sscat_kernel(dst_hbm, e16_hbm,
                  s_out,
                  idx_d, e16b, e128, zrow, s_acc, sem):
    cid = lax.axis_index("c")
    sid = lax.axis_index("s")
    wid = sid * NC + cid

    zf = jnp.zeros((16,), jnp.float32)
    for i in range(16):
        for j in range(G // 16):
            zrow[i, pl.ds(j * 16, 16)] = zf
    base_r = sid * RPT

    def zb(t, _):
        pltpu.sync_copy(zrow, s_acc.at[pl.ds(base_r + t * 16, 16)])
        return 0

    lax.fori_loop(0, RPT // 16, zb, 0)
    plsc.subcore_barrier()

    def chunk(ch, _):
        base = wid * EPW + ch * CH
        pltpu.sync_copy(dst_hbm.at[pl.ds(base, CH)], idx_d.at[0])
        pltpu.sync_copy(e16_hbm.at[pl.ds(base, CH)], e16b)

        def edge(i, _):
            eb = e16b[i]
            for j in range(G // 16):
                e128[i, pl.ds(j * 16, 16)] = eb
            return 0

        lax.fori_loop(0, CH, edge, 0)
        pltpu.sync_copy(e128, s_acc.at[idx_d.at[0]], add=True)
        return 0

    lax.fori_loop(0, NCHUNK, chunk, 0)
    plsc.subcore_barrier()
    pltpu.sync_copy(s_acc.at[pl.ds(base_r, RPT)],
                    s_out.at[cid, pl.ds(base_r, RPT)])


BRV = 512
NBV = V_PAD // BRV
BRE = 32
ER = E_PAD // 128
NBE = ER // BRE


def _full(shape):
    return pl.BlockSpec(shape, lambda i: tuple(0 for _ in shape))


def _rows(bs, *rest):
    return pl.BlockSpec((bs,) + rest, lambda i: (i,) + tuple(0 for _ in rest))


def _node_a_body(hv_ref, pnwt_ref, pnb_ref, w1nt_ref, w2c_ref,
                 hvnew_ref, hvp1_ref, sn_ref):
    x = hv_ref[...]
    hn = x @ pnwt_ref[...] + pnb_ref[...]
    hn = jnp.where(hn > 0, hn, 0.01 * hn)
    hvnew_ref[...] = hn
    hvp1_ref[...] = x @ w1nt_ref[...]
    sn_ref[...] = hn @ w2c_ref[...]


_node_a = pl.pallas_call(
    _node_a_body,
    grid=(NBV,),
    in_specs=[_rows(BRV, D), _full((D, G)), _full((1, G)), _full((D, G)),
              _full((G, G))],
    out_specs=(_rows(BRV, G), _rows(BRV, G), _rows(BRV, G)),
    out_shape=(jax.ShapeDtypeStruct((V_PAD, G), jnp.float32),
               jax.ShapeDtypeStruct((V_PAD, G), jnp.float32),
               jax.ShapeDtypeStruct((V_PAD, G), jnp.float32)),
)


def _efp_body(ef_ref, w_ref, b_ref, out_ref):
    out_ref[...] = ef_ref[...] @ w_ref[...] + b_ref[...]


_efp_k = pl.pallas_call(
    _efp_body,
    grid=(NBE,),
    in_specs=[_rows(BRE * 128, DE), _full((DE, G)), _full((1, G))],
    out_specs=_rows(BRE * 128, G),
    out_shape=jax.ShapeDtypeStruct((E_PAD, G), jnp.float32),
)


def _elogit_body(he1_ref, sdn_ref, w2e_ref, b2_ref, lg_ref, bmax_ref):
    t = jnp.sum(he1_ref[...] * w2e_ref[...][None], axis=2)
    x = sdn_ref[...] + t + b2_ref[...]
    lg = jnp.where(x > 0, x, 0.01 * x)
    lg_ref[...] = lg
    bmax_ref[...] = jnp.max(lg, axis=0, keepdims=True)[None]


_elogit = pl.pallas_call(
    _elogit_body,
    grid=(NBE,),
    in_specs=[_rows(BRE, 128, G), _rows(BRE, 128), _full((1, G)),
              _full((1, 128))],
    out_specs=(_rows(BRE, 128), _rows(1, 1, 128)),
    out_shape=(jax.ShapeDtypeStruct((ER, 128), jnp.float32),
               jax.ShapeDtypeStruct((NBE, 1, 128), jnp.float32)),
)


def _escale_body(lg_ref, m_ref, e_ref):
    e_ref[...] = jnp.exp(lg_ref[...] - m_ref[...])


_escale = pl.pallas_call(
    _escale_body,
    grid=(NBE,),
    in_specs=[_rows(BRE, 128), _full((1, 128))],
    out_specs=_rows(BRE, 128),
    out_shape=jax.ShapeDtypeStruct((ER, 128), jnp.float32),
)


def _elayer_body(d_ref, s_ref, b_ref, m_ref, e_ref):
    x = d_ref[...] + s_ref[...] + b_ref[...]
    lg = jnp.where(x > 0, x, 0.01 * x)
    e_ref[...] = jnp.exp(lg - m_ref[...])


_elayer = pl.pallas_call(
    _elayer_body,
    grid=(NBE,),
    in_specs=[_rows(BRE, 128), _rows(BRE, 128), _full((1, 128)),
              _full((1, 128))],
    out_specs=_rows(BRE, 128),
    out_shape=jax.ShapeDtypeStruct((ER, 128), jnp.float32),
)


def _gru_block(ctx, h, wih_t, whh_t, bih, bhh):
    gi = ctx @ wih_t + bih
    gh = h @ whh_t + bhh
    r = jax.nn.sigmoid(gi[:, :G] + gh[:, :G])
    z = jax.nn.sigmoid(gi[:, G:2 * G] + gh[:, G:2 * G])
    n = jnp.tanh(gi[:, 2 * G:] + r * gh[:, 2 * G:])
    node = (1.0 - z) * n + z * h
    return jnp.maximum(node, 0.0)


def _tables_block(node, packw, pnwt, pnb, scol_ref, hvproj_ref, bmax_ref):
    scol = node @ packw
    scol_ref[...] = scol
    hvproj_ref[...] = node @ pnwt + pnb
    bmax_ref[...] = jnp.max(scol, axis=0, keepdims=True)[None]


def _comb_gc_body(agg0_ref, agg1_ref, s16_ref, hvnew_ref, etwt_ref, etb_ref,
                  wih_ref, whh_ref, bih_ref, bhh_ref,
                  packw_ref, pnwt_ref, pnb_ref,
                  node_ref, scol_ref, hvproj_ref, bmax_ref):
    agg = agg0_ref[...] + agg1_ref[...]
    s = s16_ref[...][:, 0:1]
    denom = s + 1e-9
    c = (agg @ etwt_ref[...]) / denom + (s / denom) * etb_ref[...]
    ctx = jnp.where(c > 0, c, jnp.exp(c) - 1.0)
    node = _gru_block(ctx, hvnew_ref[...], wih_ref[...], whh_ref[...],
                      bih_ref[...], bhh_ref[...])
    node_ref[...] = node
    _tables_block(node, packw_ref[...], pnwt_ref[...], pnb_ref[...],
                  scol_ref, hvproj_ref, bmax_ref)


_comb_gc = pl.pallas_call(
    _comb_gc_body,
    grid=(NBV,),
    in_specs=[_rows(BRV, G), _rows(BRV, G), _rows(BRV, 16), _rows(BRV, G),
              _full((G, G)), _full((1, G)),
              _full((G, 3 * G)), _full((G, 3 * G)), _full((1, 3 * G)),
              _full((1, 3 * G)),
              _full((G, G)), _full((G, G)), _full((1, G))],
    out_specs=(_rows(BRV, G), _rows(BRV, G), _rows(BRV, G),
               _rows(1, 1, 128)),
    out_shape=(jax.ShapeDtypeStruct((V_PAD, G), jnp.float32),
               jax.ShapeDtypeStruct((V_PAD, G), jnp.float32),
               jax.ShapeDtypeStruct((V_PAD, G), jnp.float32),
               jax.ShapeDtypeStruct((NBV, 1, 128), jnp.float32)),
)


def _comb_layer_body(agg0_ref, agg1_ref, s16_ref, h_ref,
                     wih_ref, whh_ref, bih_ref, bhh_ref,
                     packw_ref, pnwt_ref, pnb_ref,
                     node_ref, scol_ref, hvproj_ref, bmax_ref):
    agg = agg0_ref[...] + agg1_ref[...]
    s = s16_ref[...][:, 0:1]
    c = agg / (s + 1e-9)
    ctx = jnp.where(c > 0, c, jnp.exp(c) - 1.0)
    node = _gru_block(ctx, h_ref[...], wih_ref[...], whh_ref[...],
                      bih_ref[...], bhh_ref[...])
    node_ref[...] = node
    _tables_block(node, packw_ref[...], pnwt_ref[...], pnb_ref[...],
                  scol_ref, hvproj_ref, bmax_ref)


_comb_layer = pl.pallas_call(
    _comb_layer_body,
    grid=(NBV,),
    in_specs=[_rows(BRV, G), _rows(BRV, G), _rows(BRV, 16), _rows(BRV, G),
              _full((G, 3 * G)), _full((G, 3 * G)), _full((1, 3 * G)),
              _full((1, 3 * G)),
              _full((G, G)), _full((G, G)), _full((1, G))],
    out_specs=(_rows(BRV, G), _rows(BRV, G), _rows(BRV, G),
               _rows(1, 1, 128)),
    out_shape=(jax.ShapeDtypeStruct((V_PAD, G), jnp.float32),
               jax.ShapeDtypeStruct((V_PAD, G), jnp.float32),
               jax.ShapeDtypeStruct((V_PAD, G), jnp.float32),
               jax.ShapeDtypeStruct((NBV, 1, 128), jnp.float32)),
)


def _leaky(x):
    return jax.nn.leaky_relu(x, negative_slope=0.01)


def _gru_update(x, h, W_ih, W_hh, b_ih, b_hh):
    gi = x @ W_ih.T + b_ih
    gh = h @ W_hh.T + b_hh
    i_r, i_z, i_n = jnp.split(gi, 3, axis=1)
    h_r, h_z, h_n = jnp.split(gh, 3, axis=1)
    r = jax.nn.sigmoid(i_r + h_r)
    z = jax.nn.sigmoid(i_z + h_z)
    n = jnp.tanh(i_n + r * h_n)
    return (1.0 - z) * n + z * h


def kernel(node_feats, edge_feats, edge_index,
           gc_pn_W, gc_pn_b, gc_pe1_W, gc_pe1_b, gc_pe2_W, gc_pe2_b,
           gc_et_W, gc_et_b, gc_gru_Wih, gc_gru_Whh, gc_gru_bih, gc_gru_bhh,
           l0_pe_W, l0_pe_b, l0_pn_W, l0_pn_b,
           l0_gru_Wih, l0_gru_Whh, l0_gru_bih, l0_gru_bhh,
           l1_pe_W, l1_pe_b, l1_pn_W, l1_pn_b,
           l1_gru_Wih, l1_gru_Whh, l1_gru_bih, l1_gru_bhh,
           pred_W, pred_b):
    f32 = jnp.float32
    src = jnp.full((E_PAD,), DUMMY, jnp.int32).at[:E].set(
        edge_index[0].astype(jnp.int32))
    dst = jnp.full((E_PAD,), DUMMY, jnp.int32).at[:E].set(
        edge_index[1].astype(jnp.int32))
    hv_pad = jnp.zeros((V_PAD, D), f32).at[:V].set(node_feats)
    ef_pad = jnp.zeros((E_PAD, DE), f32).at[:E].set(edge_feats)
    eid = jnp.arange(E_PAD, dtype=jnp.int32)

    def col_mat(*cols):
        w = jnp.zeros((G, G), f32)
        for k, c in enumerate(cols):
            w = w.at[:, k].set(c)
        return w

    hv_new, hv_p1_pad, sn_mat = _node_a(
        hv_pad, gc_pn_W.T, gc_pn_b[None], gc_pe1_W[:, :D].T,
        col_mat(gc_pe2_W[0, :G]))
    sn_pad = sn_mat[:, 0]
    efp_pad = _efp_k(ef_pad, gc_pe1_W[:, D:].T, gc_pe1_b[None])

    he1_pad, sdn = _g1_kernel(src, dst, hv_p1_pad, sn_pad, efp_pad)

    he1_3d = he1_pad.reshape(ER, 128, G)
    lg2, bmax = _elogit(he1_3d, sdn.reshape(ER, 128),
                        gc_pe2_W[0:1, G:], jnp.full((1, 128), gc_pe2_b[0]))
    M = jnp.max(bmax)
    e2 = _escale(lg2, jnp.full((1, 128), M))
    e16 = jnp.broadcast_to(e2.reshape(E_PAD)[:, None], (E_PAD, 16))

    agg2c = _scat_kernel(eid, dst, he1_pad, e16)
    s2c = _sscat_kernel(dst, e16)
    s16sum = (s2c[0] + s2c[1])[:, :16]

    node, scol, hvproj, bmax = _comb_gc(
        agg2c[0], agg2c[1], s16sum, hv_new, gc_et_W.T, gc_et_b[None],
        gc_gru_Wih.T, gc_gru_Whh.T, gc_gru_bih[None], gc_gru_bhh[None],
        col_mat(l0_pe_W[0, :G], l0_pe_W[0, G:]), l0_pn_W.T, l0_pn_b[None])

    layer_w = (
        (l0_pe_b, l0_gru_Wih, l0_gru_Whh, l0_gru_bih, l0_gru_bhh,
         col_mat(l1_pe_W[0, :G], l1_pe_W[0, G:]), l1_pn_W.T, l1_pn_b[None]),
        (l1_pe_b, l1_gru_Wih, l1_gru_Whh, l1_gru_bih, l1_gru_bhh,
         col_mat(pred_W[0]), jnp.zeros((G, G), f32), jnp.zeros((1, G), f32)),
    )
    for (pe_b, Wih, Whh, bih, bhh, next_packw, next_pnwt, next_pnb) in layer_w:
        b = pe_b[0]
        Mub = _leaky(jnp.max(bmax[:, 0, 0]) + jnp.max(bmax[:, 0, 1]) + b)
        dsc_pad = scol[:, 0]
        ssc_pad = scol[:, 1]

        d_e, s_e = _gat2_kernel(src, dst, dsc_pad, ssc_pad)
        e2 = _elayer(d_e.reshape(ER, 128), s_e.reshape(ER, 128),
                     jnp.full((1, 128), b), jnp.full((1, 128), Mub))
        e16 = jnp.broadcast_to(e2.reshape(E_PAD)[:, None], (E_PAD, 16))

        agg2c = _scat_kernel(src, dst, hvproj, e16)
        s2c = _sscat_kernel(dst, e16)
        s16sum = (s2c[0] + s2c[1])[:, :16]
        node, scol, hvproj, bmax = _comb_layer(
            agg2c[0], agg2c[1], s16sum, node,
            Wih.T, Whh.T, bih[None], bhh[None],
            next_packw, next_pnwt, next_pnb)

    return scol[:V, 0:1] + pred_b


# speedup vs baseline: 4.4492x; 1.0349x over previous
import functools

import jax
import jax.numpy as jnp
from jax import lax
from jax.experimental import pallas as pl
from jax.experimental.pallas import tpu as pltpu
from jax.experimental.pallas import tpu_sc as plsc

V = 10000
E = 320000
D = 128
DE = 16
G = 128

NC = 2
NS = 16
NW = NC * NS
V_PAD = 10240
E_PAD = 327680
EPW = E_PAD // NW
CH = 128
NCHUNK = EPW // CH
CHS = 64
NCHS = EPW // CHS
RPT = V_PAD // NS
DUMMY = V_PAD - 1

_mesh = plsc.VectorSubcoreMesh(core_axis_name="c", subcore_axis_name="s")


def _leaky_v(x):
    return jnp.where(x > 0, x, x * 0.01)


def _wid():
    return lax.axis_index("s") * NC + lax.axis_index("c")


@functools.partial(
    pl.kernel,
    out_type=(
        jax.ShapeDtypeStruct((E_PAD, G), jnp.float32),
        jax.ShapeDtypeStruct((E_PAD,), jnp.float32),
    ),
    mesh=_mesh,
    scratch_types=[
        [pltpu.VMEM((CH,), jnp.int32)] * 2,
        [pltpu.VMEM((CH,), jnp.int32)] * 2,
        [pltpu.VMEM((CH, G), jnp.float32)] * 2,
        [pltpu.VMEM((CH, G), jnp.float32)] * 2,
        [pltpu.VMEM((CH,), jnp.float32)] * 2,
        [pltpu.SemaphoreType.DMA] * 2,
    ],
)
def _g1_kernel(src_hbm, dst_hbm, hvp1_hbm, sn_hbm, efp_hbm,
               he1_hbm, sdn_hbm,
               idx_s, idx_d, rows, efb, dnv, sem):
    wid = _wid()

    def fire(ch, buf):
        base = wid * EPW + ch * CH
        pltpu.sync_copy(src_hbm.at[pl.ds(base, CH)], idx_s[buf])
        pltpu.sync_copy(dst_hbm.at[pl.ds(base, CH)], idx_d[buf])
        pltpu.async_copy(efp_hbm.at[pl.ds(base, CH)], efb[buf], sem[buf])
        pltpu.async_copy(hvp1_hbm.at[idx_s[buf]], rows[buf], sem[buf])
        pltpu.async_copy(sn_hbm.at[idx_d[buf]], dnv[buf], sem[buf])

    def finish(ch, buf):
        base = wid * EPW + ch * CH
        pltpu.make_async_copy(efp_hbm.at[pl.ds(base, CH)], efb[buf],
                              sem[buf]).wait()
        pltpu.make_async_copy(hvp1_hbm.at[idx_s[buf]], rows[buf],
                              sem[buf]).wait()
        pltpu.make_async_copy(sn_hbm.at[idx_d[buf]], dnv[buf],
                              sem[buf]).wait()
        r = rows[buf]
        ef = efb[buf]

        def edge(g, _):
            for il in range(4):
                i = g * 4 + il
                for j in range(G // 16):
                    u = r[i, pl.ds(j * 16, 16)] + ef[i, pl.ds(j * 16, 16)]
                    r[i, pl.ds(j * 16, 16)] = _leaky_v(u)
            return 0

        lax.fori_loop(0, CH // 4, edge, 0)
        pltpu.sync_copy(r, he1_hbm.at[pl.ds(base, CH)])
        pltpu.sync_copy(dnv[buf], sdn_hbm.at[pl.ds(base, CH)])

    fire(0, 0)

    def body(k, _):
        fire(2 * k + 1, 1)
        finish(2 * k, 0)

        @pl.when(k < NCHUNK // 2 - 1)
        def _():
            fire(2 * k + 2, 0)

        finish(2 * k + 1, 1)
        return 0

    lax.fori_loop(0, NCHUNK // 2, body, 0)


@functools.partial(
    pl.kernel,
    out_type=(
        jax.ShapeDtypeStruct((E_PAD,), jnp.float32),
        jax.ShapeDtypeStruct((E_PAD,), jnp.float32),
    ),
    mesh=_mesh,
    scratch_types=[
        [pltpu.VMEM((CH,), jnp.int32)] * 2,
        [pltpu.VMEM((CH,), jnp.int32)] * 2,
        [pltpu.VMEM((CH,), jnp.float32)] * 2,
        [pltpu.VMEM((CH,), jnp.float32)] * 2,
        [pltpu.SemaphoreType.DMA] * 2,
    ],
)
def _gat2_kernel(src_hbm, dst_hbm, dsc_hbm, ssc_hbm,
                 de_hbm, se_hbm,
                 idx_s, idx_d, dval, sval, sem):
    wid = _wid()

    def fire(ch, buf):
        base = wid * EPW + ch * CH
        pltpu.sync_copy(src_hbm.at[pl.ds(base, CH)], idx_s[buf])
        pltpu.sync_copy(dst_hbm.at[pl.ds(base, CH)], idx_d[buf])
        pltpu.async_copy(dsc_hbm.at[idx_d[buf]], dval[buf], sem[buf])
        pltpu.async_copy(ssc_hbm.at[idx_s[buf]], sval[buf], sem[buf])

    def finish(ch, buf):
        base = wid * EPW + ch * CH
        pltpu.make_async_copy(dsc_hbm.at[idx_d[buf]], dval[buf],
                              sem[buf]).wait()
        pltpu.make_async_copy(ssc_hbm.at[idx_s[buf]], sval[buf],
                              sem[buf]).wait()
        pltpu.sync_copy(dval[buf], de_hbm.at[pl.ds(base, CH)])
        pltpu.sync_copy(sval[buf], se_hbm.at[pl.ds(base, CH)])

    fire(0, 0)

    def body(k, _):
        fire(2 * k + 1, 1)
        finish(2 * k, 0)
        fire(2 * k + 2, 0)
        finish(2 * k + 1, 1)
        return 0

    lax.fori_loop(0, NCHUNK // 2 - 1, body, 0)
    fire(NCHUNK - 1, 1)
    finish(NCHUNK - 2, 0)
    finish(NCHUNK - 1, 1)


@functools.partial(
    pl.kernel,
    out_type=jax.ShapeDtypeStruct((NC, V_PAD, G), jnp.float32),
    mesh=_mesh,
    scratch_types=[
        [pltpu.VMEM((CHS,), jnp.int32)] * 2,
        [pltpu.VMEM((1, CHS), jnp.int32)] * 2,
        [pltpu.VMEM((CHS, G), jnp.float32)] * 2,
        [pltpu.VMEM((CHS, 16), jnp.float32)] * 2,
        pltpu.VMEM((16, G), jnp.float32),
        pltpu.VMEM_SHARED((V_PAD, G), jnp.float32),
        [pltpu.SemaphoreType.DMA] * 2,
    ],
)
def _scat_kernel(rid_hbm, dst_hbm, tab_hbm, e16_hbm,
                 agg_out,
                 idx_s, idx_d, rows, e16b, zrow, agg_acc, sem):
    cid = lax.axis_index("c")
    sid = lax.axis_index("s")
    wid = sid * NC + cid

    zf = jnp.zeros((16,), jnp.float32)
    for i in range(16):
        for j in range(G // 16):
            zrow[i, pl.ds(j * 16, 16)] = zf
    base_r = sid * RPT

    def zb(t, _):
        pltpu.sync_copy(zrow, agg_acc.at[pl.ds(base_r + t * 16, 16)])
        return 0

    lax.fori_loop(0, RPT // 16, zb, 0)
    plsc.subcore_barrier()

    def fire(ch, buf):
        base = wid * EPW + ch * CHS
        pltpu.sync_copy(rid_hbm.at[pl.ds(base, CHS)], idx_s[buf])
        pltpu.sync_copy(dst_hbm.at[pl.ds(base, CHS)], idx_d[buf].at[0])
        pltpu.sync_copy(e16_hbm.at[pl.ds(base, CHS)], e16b[buf])
        pltpu.async_copy(tab_hbm.at[idx_s[buf]], rows[buf], sem[buf])

    def finish(ch, buf):
        pltpu.make_async_copy(tab_hbm.at[idx_s[buf]], rows[buf],
                              sem[buf]).wait()
        r = rows[buf]
        ev = e16b[buf]

        def edge(g, _):
            for il in range(4):
                i = g * 4 + il
                eb = ev[i]
                for j in range(G // 16):
                    r[i, pl.ds(j * 16, 16)] = r[i, pl.ds(j * 16, 16)] * eb
            return 0

        lax.fori_loop(0, CHS // 4, edge, 0)
        pltpu.sync_copy(r, agg_acc.at[idx_d[buf].at[0]], add=True)

    fire(0, 0)

    def body(k, _):
        fire(2 * k + 1, 1)
        finish(2 * k, 0)
        fire(2 * k + 2, 0)
        finish(2 * k + 1, 1)
        return 0

    lax.fori_loop(0, NCHS // 2 - 1, body, 0)
    fire(NCHS - 1, 1)
    finish(NCHS - 2, 0)
    finish(NCHS - 1, 1)
    plsc.subcore_barrier()
    pltpu.sync_copy(agg_acc.at[pl.ds(base_r, RPT)],
                    agg_out.at[cid, pl.ds(base_r, RPT)])


@functools.partial(
    pl.kernel,
    out_type=jax.ShapeDtypeStruct((NC, V_PAD, G), jnp.float32),
    mesh=_mesh,
    scratch_types=[
        [pltpu.VMEM((1, CH), jnp.int32)] * 1,
        [pltpu.VMEM((CH, G), jnp.float32)] * 1,
        [pltpu.VMEM((CH, 16), jnp.float32)] * 1,
        pltpu.VMEM((16, G), jnp.float32),
        pltpu.VMEM_SHARED((V_PAD, G), jnp.float32),
        [pltpu.SemaphoreType.DMA] * 1,
    ],
)
def _scatseq_kernel(dst_hbm, tab_hbm,
                    agg_out,
                    idx_d, rows, e16b, zrow, agg_acc, sem):
    cid = lax.axis_index("c")
    sid = lax.axis_index("s")
    wid = sid * NC + cid

    zf = jnp.zeros((16,), jnp.float32)
    for i in range(16):
        for j in range(G // 16):
            zrow[i, pl.ds(j * 16, 16)] = zf
    base_r = sid * RPT

    def zb(t, _):
        pltpu.sync_copy(zrow, agg_acc.at[pl.ds(base_r + t * 16, 16)])
        return 0

    lax.fori_loop(0, RPT // 16, zb, 0)
    plsc.subcore_barrier()

    def chunk(ch, _):
        base = wid * EPW + ch * CH
        pltpu.sync_copy(dst_hbm.at[pl.ds(base, CH)], idx_d[0].at[0])
        pltpu.sync_copy(tab_hbm.at[pl.ds(base, CH)], rows[0])
        pltpu.sync_copy(rows[0], agg_acc.at[idx_d[0].at[0]], add=True)
        return 0

    lax.fori_loop(0, NCHUNK, chunk, 0)
    plsc.subcore_barrier()
    pltpu.sync_copy(agg_acc.at[pl.ds(base_r, RPT)],
                    agg_out.at[cid, pl.ds(base_r, RPT)])


@functools.partial(
    pl.kernel,
    out_type=jax.ShapeDtypeStruct((NC, V_PAD, G), jnp.float32),
    mesh=_mesh,
    scratch_types=[
        pltpu.VMEM((1, CH), jnp.int32),
        pltpu.VMEM((CH, 16), jnp.float32),
        pltpu.VMEM((CH, G), jnp.float32),
        pltpu.VMEM((16, G), jnp.float32),
        pltpu.VMEM_SHARED((V_PAD, G), jnp.float32),
        pltpu.SemaphoreType.DMA,
    ],
)
def _sscat_kernel(dst_hbm, e16_hbm,
                  s_out,
                  idx_d, e16b, e128, zrow, s_acc, sem):
    cid = lax.axis_index("c")
    sid = lax.axis_index("s")
    wid = sid * NC + cid

    zf = jnp.zeros((16,), jnp.float32)
    for i in range(16):
        for j in range(G // 16):
            zrow[i, pl.ds(j * 16, 16)] = zf
    base_r = sid * RPT

    def zb(t, _):
        pltpu.sync_copy(zrow, s_acc.at[pl.ds(base_r + t * 16, 16)])
        return 0

    lax.fori_loop(0, RPT // 16, zb, 0)
    plsc.subcore_barrier()

    def chunk(ch, _):
        base = wid * EPW + ch * CH
        pltpu.sync_copy(dst_hbm.at[pl.ds(base, CH)], idx_d.at[0])
        pltpu.sync_copy(e16_hbm.at[pl.ds(base, CH)], e16b)

        def edge(g, _):
            for il in range(4):
                i = g * 4 + il
                eb = e16b[i]
                for j in range(G // 16):
                    e128[i, pl.ds(j * 16, 16)] = eb
            return 0

        lax.fori_loop(0, CH // 4, edge, 0)
        pltpu.sync_copy(e128, s_acc.at[idx_d.at[0]], add=True)
        return 0

    lax.fori_loop(0, NCHUNK, chunk, 0)
    plsc.subcore_barrier()
    pltpu.sync_copy(s_acc.at[pl.ds(base_r, RPT)],
                    s_out.at[cid, pl.ds(base_r, RPT)])


BRV = 512
NBV = V_PAD // BRV
BRE = 32
ER = E_PAD // 128
NBE = ER // BRE


def _full(shape):
    return pl.BlockSpec(shape, lambda i: tuple(0 for _ in shape))


def _rows(bs, *rest):
    return pl.BlockSpec((bs,) + rest, lambda i: (i,) + tuple(0 for _ in rest))


def _node_a_body(hv_ref, pnwt_ref, pnb_ref, w1nt_ref, w2c_ref,
                 hvnew_ref, hvp1_ref, sn_ref):
    x = hv_ref[...]
    hn = x @ pnwt_ref[...] + pnb_ref[...]
    hn = jnp.where(hn > 0, hn, 0.01 * hn)
    hvnew_ref[...] = hn
    hvp1_ref[...] = x @ w1nt_ref[...]
    sn_ref[...] = hn @ w2c_ref[...]


_node_a = pl.pallas_call(
    _node_a_body,
    grid=(NBV,),
    in_specs=[_rows(BRV, D), _full((D, G)), _full((1, G)), _full((D, G)),
              _full((G, G))],
    out_specs=(_rows(BRV, G), _rows(BRV, G), _rows(BRV, G)),
    out_shape=(jax.ShapeDtypeStruct((V_PAD, G), jnp.float32),
               jax.ShapeDtypeStruct((V_PAD, G), jnp.float32),
               jax.ShapeDtypeStruct((V_PAD, G), jnp.float32)),
)


def _efp_body(ef_ref, w_ref, b_ref, out_ref):
    out_ref[...] = ef_ref[...] @ w_ref[...] + b_ref[...]


_efp_k = pl.pallas_call(
    _efp_body,
    grid=(NBE,),
    in_specs=[_rows(BRE * 128, DE), _full((DE, G)), _full((1, G))],
    out_specs=_rows(BRE * 128, G),
    out_shape=jax.ShapeDtypeStruct((E_PAD, G), jnp.float32),
)


def _elogit_body(he1_ref, sdn_ref, w2e_ref, b2_ref, lg_ref, bmax_ref):
    t = jnp.sum(he1_ref[...] * w2e_ref[...][None], axis=2)
    x = sdn_ref[...] + t + b2_ref[...]
    lg = jnp.where(x > 0, x, 0.01 * x)
    lg_ref[...] = lg
    bmax_ref[...] = jnp.max(lg, axis=0, keepdims=True)[None]


_elogit = pl.pallas_call(
    _elogit_body,
    grid=(NBE,),
    in_specs=[_rows(BRE, 128, G), _rows(BRE, 128), _full((1, G)),
              _full((1, 128))],
    out_specs=(_rows(BRE, 128), _rows(1, 1, 128)),
    out_shape=(jax.ShapeDtypeStruct((ER, 128), jnp.float32),
               jax.ShapeDtypeStruct((NBE, 1, 128), jnp.float32)),
)


def _escale_body(lg_ref, m_ref, he1_ref, e_ref, sc_ref):
    e = jnp.exp(lg_ref[...] - m_ref[...])
    e_ref[...] = e
    sc_ref[...] = he1_ref[...] * e[:, :, None]


_escale = pl.pallas_call(
    _escale_body,
    grid=(NBE,),
    in_specs=[_rows(BRE, 128), _full((1, 128)), _rows(BRE, 128, G)],
    out_specs=(_rows(BRE, 128), _rows(BRE, 128, G)),
    out_shape=(jax.ShapeDtypeStruct((ER, 128), jnp.float32),
               jax.ShapeDtypeStruct((ER, 128, G), jnp.float32)),
)


def _elayer_body(d_ref, s_ref, b_ref, m_ref, e_ref):
    x = d_ref[...] + s_ref[...] + b_ref[...]
    lg = jnp.where(x > 0, x, 0.01 * x)
    e_ref[...] = jnp.exp(lg - m_ref[...])


_elayer = pl.pallas_call(
    _elayer_body,
    grid=(NBE,),
    in_specs=[_rows(BRE, 128), _rows(BRE, 128), _full((1, 128)),
              _full((1, 128))],
    out_specs=_rows(BRE, 128),
    out_shape=jax.ShapeDtypeStruct((ER, 128), jnp.float32),
)


def _gru_block(ctx, h, wih_t, whh_t, bih, bhh):
    gi = ctx @ wih_t + bih
    gh = h @ whh_t + bhh
    r = jax.nn.sigmoid(gi[:, :G] + gh[:, :G])
    z = jax.nn.sigmoid(gi[:, G:2 * G] + gh[:, G:2 * G])
    n = jnp.tanh(gi[:, 2 * G:] + r * gh[:, 2 * G:])
    node = (1.0 - z) * n + z * h
    return jnp.maximum(node, 0.0)


def _tables_block(node, packw, pnwt, pnb, scol_ref, hvproj_ref, bmax_ref):
    scol = node @ packw
    scol_ref[...] = scol
    hvproj_ref[...] = node @ pnwt + pnb
    bmax_ref[...] = jnp.max(scol, axis=0, keepdims=True)[None]


def _comb_gc_body(agg0_ref, agg1_ref, s16_ref, hvnew_ref, etwt_ref, etb_ref,
                  wih_ref, whh_ref, bih_ref, bhh_ref,
                  packw_ref, pnwt_ref, pnb_ref,
                  node_ref, scol_ref, hvproj_ref, bmax_ref):
    agg = agg0_ref[...] + agg1_ref[...]
    s = s16_ref[...][:, 0:1]
    denom = s + 1e-9
    c = (agg @ etwt_ref[...]) / denom + (s / denom) * etb_ref[...]
    ctx = jnp.where(c > 0, c, jnp.exp(c) - 1.0)
    node = _gru_block(ctx, hvnew_ref[...], wih_ref[...], whh_ref[...],
                      bih_ref[...], bhh_ref[...])
    node_ref[...] = node
    _tables_block(node, packw_ref[...], pnwt_ref[...], pnb_ref[...],
                  scol_ref, hvproj_ref, bmax_ref)


_comb_gc = pl.pallas_call(
    _comb_gc_body,
    grid=(NBV,),
    in_specs=[_rows(BRV, G), _rows(BRV, G), _rows(BRV, 16), _rows(BRV, G),
              _full((G, G)), _full((1, G)),
              _full((G, 3 * G)), _full((G, 3 * G)), _full((1, 3 * G)),
              _full((1, 3 * G)),
              _full((G, G)), _full((G, G)), _full((1, G))],
    out_specs=(_rows(BRV, G), _rows(BRV, G), _rows(BRV, G),
               _rows(1, 1, 128)),
    out_shape=(jax.ShapeDtypeStruct((V_PAD, G), jnp.float32),
               jax.ShapeDtypeStruct((V_PAD, G), jnp.float32),
               jax.ShapeDtypeStruct((V_PAD, G), jnp.float32),
               jax.ShapeDtypeStruct((NBV, 1, 128), jnp.float32)),
)


def _comb_layer_body(agg0_ref, agg1_ref, s16_ref, h_ref,
                     wih_ref, whh_ref, bih_ref, bhh_ref,
                     packw_ref, pnwt_ref, pnb_ref,
                     node_ref, scol_ref, hvproj_ref, bmax_ref):
    agg = agg0_ref[...] + agg1_ref[...]
    s = s16_ref[...][:, 0:1]
    c = agg / (s + 1e-9)
    ctx = jnp.where(c > 0, c, jnp.exp(c) - 1.0)
    node = _gru_block(ctx, h_ref[...], wih_ref[...], whh_ref[...],
                      bih_ref[...], bhh_ref[...])
    node_ref[...] = node
    _tables_block(node, packw_ref[...], pnwt_ref[...], pnb_ref[...],
                  scol_ref, hvproj_ref, bmax_ref)


_comb_layer = pl.pallas_call(
    _comb_layer_body,
    grid=(NBV,),
    in_specs=[_rows(BRV, G), _rows(BRV, G), _rows(BRV, 16), _rows(BRV, G),
              _full((G, 3 * G)), _full((G, 3 * G)), _full((1, 3 * G)),
              _full((1, 3 * G)),
              _full((G, G)), _full((G, G)), _full((1, G))],
    out_specs=(_rows(BRV, G), _rows(BRV, G), _rows(BRV, G),
               _rows(1, 1, 128)),
    out_shape=(jax.ShapeDtypeStruct((V_PAD, G), jnp.float32),
               jax.ShapeDtypeStruct((V_PAD, G), jnp.float32),
               jax.ShapeDtypeStruct((V_PAD, G), jnp.float32),
               jax.ShapeDtypeStruct((NBV, 1, 128), jnp.float32)),
)


def _leaky(x):
    return jax.nn.leaky_relu(x, negative_slope=0.01)


def _gru_update(x, h, W_ih, W_hh, b_ih, b_hh):
    gi = x @ W_ih.T + b_ih
    gh = h @ W_hh.T + b_hh
    i_r, i_z, i_n = jnp.split(gi, 3, axis=1)
    h_r, h_z, h_n = jnp.split(gh, 3, axis=1)
    r = jax.nn.sigmoid(i_r + h_r)
    z = jax.nn.sigmoid(i_z + h_z)
    n = jnp.tanh(i_n + r * h_n)
    return (1.0 - z) * n + z * h


def kernel(node_feats, edge_feats, edge_index,
           gc_pn_W, gc_pn_b, gc_pe1_W, gc_pe1_b, gc_pe2_W, gc_pe2_b,
           gc_et_W, gc_et_b, gc_gru_Wih, gc_gru_Whh, gc_gru_bih, gc_gru_bhh,
           l0_pe_W, l0_pe_b, l0_pn_W, l0_pn_b,
           l0_gru_Wih, l0_gru_Whh, l0_gru_bih, l0_gru_bhh,
           l1_pe_W, l1_pe_b, l1_pn_W, l1_pn_b,
           l1_gru_Wih, l1_gru_Whh, l1_gru_bih, l1_gru_bhh,
           pred_W, pred_b):
    f32 = jnp.float32
    src = jnp.full((E_PAD,), DUMMY, jnp.int32).at[:E].set(
        edge_index[0].astype(jnp.int32))
    dst = jnp.full((E_PAD,), DUMMY, jnp.int32).at[:E].set(
        edge_index[1].astype(jnp.int32))
    hv_pad = jnp.zeros((V_PAD, D), f32).at[:V].set(node_feats)
    ef_pad = jnp.zeros((E_PAD, DE), f32).at[:E].set(edge_feats)

    def col_mat(*cols):
        w = jnp.zeros((G, G), f32)
        for k, c in enumerate(cols):
            w = w.at[:, k].set(c)
        return w

    hv_new, hv_p1_pad, sn_mat = _node_a(
        hv_pad, gc_pn_W.T, gc_pn_b[None], gc_pe1_W[:, :D].T,
        col_mat(gc_pe2_W[0, :G]))
    sn_pad = sn_mat[:, 0]
    efp_pad = _efp_k(ef_pad, gc_pe1_W[:, D:].T, gc_pe1_b[None])

    he1_pad, sdn = _g1_kernel(src, dst, hv_p1_pad, sn_pad, efp_pad)

    he1_3d = he1_pad.reshape(ER, 128, G)
    lg2, bmax = _elogit(he1_3d, sdn.reshape(ER, 128),
                        gc_pe2_W[0:1, G:], jnp.full((1, 128), gc_pe2_b[0]))
    M = jnp.max(bmax)
    e2, she1 = _escale(lg2, jnp.full((1, 128), M), he1_3d)
    e16 = jnp.broadcast_to(e2.reshape(E_PAD)[:, None], (E_PAD, 16))

    agg2c = _scatseq_kernel(dst, she1.reshape(E_PAD, G))
    s2c = _sscat_kernel(dst, e16)
    s16sum = (s2c[0] + s2c[1])[:, :16]

    node, scol, hvproj, bmax = _comb_gc(
        agg2c[0], agg2c[1], s16sum, hv_new, gc_et_W.T, gc_et_b[None],
        gc_gru_Wih.T, gc_gru_Whh.T, gc_gru_bih[None], gc_gru_bhh[None],
        col_mat(l0_pe_W[0, :G], l0_pe_W[0, G:]), l0_pn_W.T, l0_pn_b[None])

    layer_w = (
        (l0_pe_b, l0_gru_Wih, l0_gru_Whh, l0_gru_bih, l0_gru_bhh,
         col_mat(l1_pe_W[0, :G], l1_pe_W[0, G:]), l1_pn_W.T, l1_pn_b[None]),
        (l1_pe_b, l1_gru_Wih, l1_gru_Whh, l1_gru_bih, l1_gru_bhh,
         col_mat(pred_W[0]), jnp.zeros((G, G), f32), jnp.zeros((1, G), f32)),
    )
    for (pe_b, Wih, Whh, bih, bhh, next_packw, next_pnwt, next_pnb) in layer_w:
        b = pe_b[0]
        Mub = _leaky(jnp.max(bmax[:, 0, 0]) + jnp.max(bmax[:, 0, 1]) + b)
        dsc_pad = scol[:, 0]
        ssc_pad = scol[:, 1]

        d_e, s_e = _gat2_kernel(src, dst, dsc_pad, ssc_pad)
        e2 = _elayer(d_e.reshape(ER, 128), s_e.reshape(ER, 128),
                     jnp.full((1, 128), b), jnp.full((1, 128), Mub))
        e16 = jnp.broadcast_to(e2.reshape(E_PAD)[:, None], (E_PAD, 16))

        agg2c = _scat_kernel(src, dst, hvproj, e16)
        s2c = _sscat_kernel(dst, e16)
        s16sum = (s2c[0] + s2c[1])[:, :16]
        node, scol, hvproj, bmax = _comb_layer(
            agg2c[0], agg2c[1], s16sum, node,
            Wih.T, Whh.T, bih[None], bhh[None],
            next_packw, next_pnwt, next_pnb)

    return scol[:V, 0:1] + pred_b


# speedup vs baseline: 5.2799x; 1.1867x over previous
import functools

import jax
import jax.numpy as jnp
from jax import lax
from jax.experimental import pallas as pl
from jax.experimental.pallas import tpu as pltpu
from jax.experimental.pallas import tpu_sc as plsc

V = 10000
E = 320000
D = 128
DE = 16
G = 128

NC = 2
NS = 16
NW = NC * NS
V_PAD = 10240
E_PAD = 327680
EPW = E_PAD // NW
CH = 128
NCHUNK = EPW // CH
CHS = 64
NCHS = EPW // CHS
RPT = V_PAD // NS
DUMMY = V_PAD - 1

_mesh = plsc.VectorSubcoreMesh(core_axis_name="c", subcore_axis_name="s")


def _leaky_v(x):
    return jnp.where(x > 0, x, x * 0.01)


def _wid():
    return lax.axis_index("s") * NC + lax.axis_index("c")


@functools.partial(
    pl.kernel,
    out_type=(
        jax.ShapeDtypeStruct((E_PAD, G), jnp.float32),
        jax.ShapeDtypeStruct((E_PAD,), jnp.float32),
    ),
    mesh=_mesh,
    scratch_types=[
        [pltpu.VMEM((CH,), jnp.int32)] * 2,
        [pltpu.VMEM((CH,), jnp.int32)] * 2,
        [pltpu.VMEM((CH, G), jnp.float32)] * 2,
        [pltpu.VMEM((CH, G), jnp.float32)] * 2,
        [pltpu.VMEM((CH,), jnp.float32)] * 2,
        [pltpu.SemaphoreType.DMA] * 2,
    ],
)
def _g1_kernel(src_hbm, dst_hbm, hvp1_hbm, sn_hbm, efp_hbm,
               he1_hbm, sdn_hbm,
               idx_s, idx_d, rows, efb, dnv, sem):
    wid = _wid()

    def fire(ch, buf):
        base = wid * EPW + ch * CH
        pltpu.sync_copy(src_hbm.at[pl.ds(base, CH)], idx_s[buf])
        pltpu.sync_copy(dst_hbm.at[pl.ds(base, CH)], idx_d[buf])
        pltpu.async_copy(efp_hbm.at[pl.ds(base, CH)], efb[buf], sem[buf])
        pltpu.async_copy(hvp1_hbm.at[idx_s[buf]], rows[buf], sem[buf])
        pltpu.async_copy(sn_hbm.at[idx_d[buf]], dnv[buf], sem[buf])

    def finish(ch, buf):
        base = wid * EPW + ch * CH
        pltpu.make_async_copy(efp_hbm.at[pl.ds(base, CH)], efb[buf],
                              sem[buf]).wait()
        pltpu.make_async_copy(hvp1_hbm.at[idx_s[buf]], rows[buf],
                              sem[buf]).wait()
        pltpu.make_async_copy(sn_hbm.at[idx_d[buf]], dnv[buf],
                              sem[buf]).wait()
        r = rows[buf]
        ef = efb[buf]

        def edge(g, _):
            for il in range(4):
                i = g * 4 + il
                for j in range(G // 16):
                    u = r[i, pl.ds(j * 16, 16)] + ef[i, pl.ds(j * 16, 16)]
                    r[i, pl.ds(j * 16, 16)] = _leaky_v(u)
            return 0

        lax.fori_loop(0, CH // 4, edge, 0)
        pltpu.sync_copy(r, he1_hbm.at[pl.ds(base, CH)])
        pltpu.sync_copy(dnv[buf], sdn_hbm.at[pl.ds(base, CH)])

    fire(0, 0)

    def body(k, _):
        fire(2 * k + 1, 1)
        finish(2 * k, 0)

        @pl.when(k < NCHUNK // 2 - 1)
        def _():
            fire(2 * k + 2, 0)

        finish(2 * k + 1, 1)
        return 0

    lax.fori_loop(0, NCHUNK // 2, body, 0)


@functools.partial(
    pl.kernel,
    out_type=(
        jax.ShapeDtypeStruct((E_PAD,), jnp.float32),
        jax.ShapeDtypeStruct((E_PAD,), jnp.float32),
    ),
    mesh=_mesh,
    scratch_types=[
        [pltpu.VMEM((CH,), jnp.int32)] * 2,
        [pltpu.VMEM((CH,), jnp.int32)] * 2,
        [pltpu.VMEM((CH,), jnp.float32)] * 2,
        [pltpu.VMEM((CH,), jnp.float32)] * 2,
        [pltpu.SemaphoreType.DMA] * 2,
    ],
)
def _gat2_kernel(src_hbm, dst_hbm, dsc_hbm, ssc_hbm,
                 de_hbm, se_hbm,
                 idx_s, idx_d, dval, sval, sem):
    wid = _wid()

    def fire(ch, buf):
        base = wid * EPW + ch * CH
        pltpu.sync_copy(src_hbm.at[pl.ds(base, CH)], idx_s[buf])
        pltpu.sync_copy(dst_hbm.at[pl.ds(base, CH)], idx_d[buf])
        pltpu.async_copy(dsc_hbm.at[idx_d[buf]], dval[buf], sem[buf])
        pltpu.async_copy(ssc_hbm.at[idx_s[buf]], sval[buf], sem[buf])

    def finish(ch, buf):
        base = wid * EPW + ch * CH
        pltpu.make_async_copy(dsc_hbm.at[idx_d[buf]], dval[buf],
                              sem[buf]).wait()
        pltpu.make_async_copy(ssc_hbm.at[idx_s[buf]], sval[buf],
                              sem[buf]).wait()
        pltpu.sync_copy(dval[buf], de_hbm.at[pl.ds(base, CH)])
        pltpu.sync_copy(sval[buf], se_hbm.at[pl.ds(base, CH)])

    fire(0, 0)

    def body(k, _):
        fire(2 * k + 1, 1)
        finish(2 * k, 0)
        fire(2 * k + 2, 0)
        finish(2 * k + 1, 1)
        return 0

    lax.fori_loop(0, NCHUNK // 2 - 1, body, 0)
    fire(NCHUNK - 1, 1)
    finish(NCHUNK - 2, 0)
    finish(NCHUNK - 1, 1)


@functools.partial(
    pl.kernel,
    out_type=jax.ShapeDtypeStruct((NC, V_PAD, G), jnp.float32),
    mesh=_mesh,
    scratch_types=[
        [pltpu.VMEM((CH,), jnp.int32)] * 1,
        [pltpu.VMEM((1, CH), jnp.int32)] * 1,
        [pltpu.VMEM((CH, G), jnp.float32)] * 1,
        [pltpu.VMEM((CH, 16), jnp.float32)] * 1,
        pltpu.VMEM((16, G), jnp.float32),
        pltpu.VMEM_SHARED((V_PAD, G), jnp.float32),
        [pltpu.SemaphoreType.DMA] * 1,
    ],
)
def _scat_kernel(rid_hbm, dst_hbm, tab_hbm, e16_hbm,
                 agg_out,
                 idx_s, idx_d, rows, e16b, zrow, agg_acc, sem):
    cid = lax.axis_index("c")
    sid = lax.axis_index("s")
    wid = sid * NC + cid

    zf = jnp.zeros((16,), jnp.float32)
    for i in range(16):
        for j in range(G // 16):
            zrow[i, pl.ds(j * 16, 16)] = zf
    base_r = sid * RPT

    def zb(t, _):
        pltpu.sync_copy(zrow, agg_acc.at[pl.ds(base_r + t * 16, 16)])
        return 0

    lax.fori_loop(0, RPT // 16, zb, 0)
    plsc.subcore_barrier()

    def chunk(ch, _):
        base = wid * EPW + ch * CH
        pltpu.async_copy(rid_hbm.at[pl.ds(base, CH)], idx_s[0], sem[0])
        pltpu.async_copy(dst_hbm.at[pl.ds(base, CH)], idx_d[0].at[0], sem[0])
        pltpu.async_copy(e16_hbm.at[pl.ds(base, CH)], e16b[0], sem[0])
        pltpu.make_async_copy(rid_hbm.at[pl.ds(base, CH)], idx_s[0],
                              sem[0]).wait()
        pltpu.make_async_copy(dst_hbm.at[pl.ds(base, CH)], idx_d[0].at[0],
                              sem[0]).wait()
        pltpu.make_async_copy(e16_hbm.at[pl.ds(base, CH)], e16b[0],
                              sem[0]).wait()
        pltpu.async_copy(tab_hbm.at[idx_s[0]], rows[0], sem[0]).wait()
        r = rows[0]
        ev = e16b[0]

        def edge(g, _):
            for il in range(4):
                i = g * 4 + il
                eb = ev[i]
                for j in range(G // 16):
                    r[i, pl.ds(j * 16, 16)] = r[i, pl.ds(j * 16, 16)] * eb
            return 0

        lax.fori_loop(0, CH // 4, edge, 0)
        pltpu.sync_copy(r, agg_acc.at[idx_d[0].at[0]], add=True)
        return 0

    lax.fori_loop(0, NCHUNK, chunk, 0)
    plsc.subcore_barrier()
    pltpu.sync_copy(agg_acc.at[pl.ds(base_r, RPT)],
                    agg_out.at[cid, pl.ds(base_r, RPT)])


@functools.partial(
    pl.kernel,
    out_type=jax.ShapeDtypeStruct((NC, V_PAD, G), jnp.float32),
    mesh=_mesh,
    scratch_types=[
        [pltpu.VMEM((1, CH), jnp.int32)] * 1,
        [pltpu.VMEM((CH, G), jnp.float32)] * 1,
        [pltpu.VMEM((CH, 16), jnp.float32)] * 1,
        pltpu.VMEM((16, G), jnp.float32),
        pltpu.VMEM_SHARED((V_PAD, G), jnp.float32),
        [pltpu.SemaphoreType.DMA] * 1,
    ],
)
def _scatseq_kernel(dst_hbm, tab_hbm,
                    agg_out,
                    idx_d, rows, e16b, zrow, agg_acc, sem):
    cid = lax.axis_index("c")
    sid = lax.axis_index("s")
    wid = sid * NC + cid

    zf = jnp.zeros((16,), jnp.float32)
    for i in range(16):
        for j in range(G // 16):
            zrow[i, pl.ds(j * 16, 16)] = zf
    base_r = sid * RPT

    def zb(t, _):
        pltpu.sync_copy(zrow, agg_acc.at[pl.ds(base_r + t * 16, 16)])
        return 0

    lax.fori_loop(0, RPT // 16, zb, 0)
    plsc.subcore_barrier()

    def chunk(ch, _):
        base = wid * EPW + ch * CH
        pltpu.async_copy(dst_hbm.at[pl.ds(base, CH)], idx_d[0].at[0], sem[0])
        pltpu.async_copy(tab_hbm.at[pl.ds(base, CH)], rows[0], sem[0])
        pltpu.make_async_copy(dst_hbm.at[pl.ds(base, CH)], idx_d[0].at[0],
                              sem[0]).wait()
        pltpu.make_async_copy(tab_hbm.at[pl.ds(base, CH)], rows[0],
                              sem[0]).wait()
        pltpu.sync_copy(rows[0], agg_acc.at[idx_d[0].at[0]], add=True)
        return 0

    lax.fori_loop(0, NCHUNK, chunk, 0)
    plsc.subcore_barrier()
    pltpu.sync_copy(agg_acc.at[pl.ds(base_r, RPT)],
                    agg_out.at[cid, pl.ds(base_r, RPT)])


@functools.partial(
    pl.kernel,
    out_type=jax.ShapeDtypeStruct((NC, V_PAD, G), jnp.float32),
    mesh=_mesh,
    scratch_types=[
        pltpu.VMEM((1, CH), jnp.int32),
        pltpu.VMEM((CH, 16), jnp.float32),
        pltpu.VMEM((CH, G), jnp.float32),
        pltpu.VMEM((16, G), jnp.float32),
        pltpu.VMEM_SHARED((V_PAD, G), jnp.float32),
        pltpu.SemaphoreType.DMA,
    ],
)
def _sscat_kernel(dst_hbm, e16_hbm,
                  s_out,
                  idx_d, e16b, e128, zrow, s_acc, sem):
    cid = lax.axis_index("c")
    sid = lax.axis_index("s")
    wid = sid * NC + cid

    zf = jnp.zeros((16,), jnp.float32)
    for i in range(16):
        for j in range(G // 16):
            zrow[i, pl.ds(j * 16, 16)] = zf
    base_r = sid * RPT

    def zb(t, _):
        pltpu.sync_copy(zrow, s_acc.at[pl.ds(base_r + t * 16, 16)])
        return 0

    lax.fori_loop(0, RPT // 16, zb, 0)
    plsc.subcore_barrier()

    def chunk(ch, _):
        base = wid * EPW + ch * CH
        pltpu.async_copy(dst_hbm.at[pl.ds(base, CH)], idx_d.at[0], sem)
        pltpu.async_copy(e16_hbm.at[pl.ds(base, CH)], e16b, sem)
        pltpu.make_async_copy(dst_hbm.at[pl.ds(base, CH)], idx_d.at[0],
                              sem).wait()
        pltpu.make_async_copy(e16_hbm.at[pl.ds(base, CH)], e16b, sem).wait()

        def edge(g, _):
            for il in range(4):
                i = g * 4 + il
                eb = e16b[i]
                for j in range(G // 16):
                    e128[i, pl.ds(j * 16, 16)] = eb
            return 0

        lax.fori_loop(0, CH // 4, edge, 0)
        pltpu.sync_copy(e128, s_acc.at[idx_d.at[0]], add=True)
        return 0

    lax.fori_loop(0, NCHUNK, chunk, 0)
    plsc.subcore_barrier()
    pltpu.sync_copy(s_acc.at[pl.ds(base_r, RPT)],
                    s_out.at[cid, pl.ds(base_r, RPT)])


EPT2 = E_PAD // NS
NCH2S = EPT2 // CHS
NCH2 = EPT2 // CH


def _zero_acc(zrow, acc, sid):
    zf = jnp.zeros((16,), jnp.float32)
    for i in range(16):
        for j in range(G // 16):
            zrow[i, pl.ds(j * 16, 16)] = zf
    base_r = sid * RPT

    def zb(t, _):
        pltpu.sync_copy(zrow, acc.at[pl.ds(base_r + t * 16, 16)])
        return 0

    lax.fori_loop(0, RPT // 16, zb, 0)
    return base_r


CH1 = 32


def _score1_escatter(dst_hbm, e16_hbm, s_out, idx_d, e16b, e128, acc, sid):
    def chunk(ch, _):
        base = sid * EPT2 + ch * CH1
        pltpu.sync_copy(dst_hbm.at[pl.ds(base, CH1)], idx_d.at[0])
        pltpu.sync_copy(e16_hbm.at[pl.ds(base, CH1)], e16b)

        def edge(g, _):
            for il in range(4):
                i = g * 4 + il
                eb = e16b[i]
                for j in range(G // 16):
                    e128[i, pl.ds(j * 16, 16)] = eb
            return 0

        lax.fori_loop(0, CH1 // 4, edge, 0)
        pltpu.sync_copy(e128, acc.at[idx_d.at[0]], add=True)
        return 0

    lax.fori_loop(0, EPT2 // CH1, chunk, 0)


@functools.partial(
    pl.kernel,
    out_type=(
        jax.ShapeDtypeStruct((V_PAD, G), jnp.float32),
        jax.ShapeDtypeStruct((V_PAD, G), jnp.float32),
    ),
    mesh=_mesh,
    scratch_types=[
        [pltpu.VMEM((CHS,), jnp.int32)] * 2,
        [pltpu.VMEM((1, CHS), jnp.int32)] * 2,
        [pltpu.VMEM((CHS, G), jnp.float32)] * 2,
        [pltpu.VMEM((CHS, 16), jnp.float32)] * 2,
        pltpu.VMEM((1, 32), jnp.int32),
        pltpu.VMEM((32, 16), jnp.float32),
        pltpu.VMEM((32, G), jnp.float32),
        pltpu.VMEM((16, G), jnp.float32),
        pltpu.VMEM_SHARED((V_PAD, G), jnp.float32),
        [pltpu.SemaphoreType.DMA] * 2,
    ],
)
def _dualg_kernel(src_hbm, dst_hbm, tab_hbm, e16_hbm,
                  agg_out, s_out,
                  idx_s, idx_d, rows, e16b, idx_d1, e16b1, e128, zrow, acc,
                  sem):
    cid = lax.axis_index("c")
    sid = lax.axis_index("s")
    base_r = _zero_acc(zrow, acc, sid)
    plsc.subcore_barrier()

    @pl.when(cid == 0)
    def _():
        def fire(ch, buf):
            base = sid * EPT2 + ch * CHS
            pltpu.sync_copy(src_hbm.at[pl.ds(base, CHS)], idx_s[buf])
            pltpu.sync_copy(dst_hbm.at[pl.ds(base, CHS)], idx_d[buf].at[0])
            pltpu.sync_copy(e16_hbm.at[pl.ds(base, CHS)], e16b[buf])
            pltpu.async_copy(tab_hbm.at[idx_s[buf]], rows[buf], sem[buf])

        def finish(ch, buf):
            pltpu.make_async_copy(tab_hbm.at[idx_s[buf]], rows[buf],
                                  sem[buf]).wait()
            r = rows[buf]
            ev = e16b[buf]

            def edge(g, _):
                for il in range(4):
                    i = g * 4 + il
                    eb = ev[i]
                    for j in range(G // 16):
                        r[i, pl.ds(j * 16, 16)] = r[i, pl.ds(j * 16, 16)] * eb
                return 0

            lax.fori_loop(0, CHS // 4, edge, 0)
            pltpu.sync_copy(r, acc.at[idx_d[buf].at[0]], add=True)

        fire(0, 0)

        def body(k, _):
            fire(2 * k + 1, 1)
            finish(2 * k, 0)

            @pl.when(k < NCH2S // 2 - 1)
            def _():
                fire(2 * k + 2, 0)

            finish(2 * k + 1, 1)
            return 0

        lax.fori_loop(0, NCH2S // 2, body, 0)

    @pl.when(cid == 1)
    def _():
        _score1_escatter(dst_hbm, e16_hbm, s_out, idx_d1, e16b1, e128, acc,
                         sid)

    plsc.subcore_barrier()

    @pl.when(cid == 0)
    def _():
        pltpu.sync_copy(acc.at[pl.ds(base_r, RPT)],
                        agg_out.at[pl.ds(base_r, RPT)])

    @pl.when(cid == 1)
    def _():
        pltpu.sync_copy(acc.at[pl.ds(base_r, RPT)],
                        s_out.at[pl.ds(base_r, RPT)])


@functools.partial(
    pl.kernel,
    out_type=(
        jax.ShapeDtypeStruct((V_PAD, G), jnp.float32),
        jax.ShapeDtypeStruct((V_PAD, G), jnp.float32),
    ),
    mesh=_mesh,
    scratch_types=[
        [pltpu.VMEM((1, CHS), jnp.int32)] * 2,
        [pltpu.VMEM((CHS, G), jnp.float32)] * 2,
        pltpu.VMEM((1, 32), jnp.int32),
        pltpu.VMEM((32, 16), jnp.float32),
        pltpu.VMEM((32, G), jnp.float32),
        pltpu.VMEM((16, G), jnp.float32),
        pltpu.VMEM_SHARED((V_PAD, G), jnp.float32),
        [pltpu.SemaphoreType.DMA] * 2,
    ],
)
def _dualseq_kernel(dst_hbm, tab_hbm, e16_hbm,
                    agg_out, s_out,
                    idx_d, rows, idx_d1, e16b1, e128, zrow, acc, sem):
    cid = lax.axis_index("c")
    sid = lax.axis_index("s")
    base_r = _zero_acc(zrow, acc, sid)
    plsc.subcore_barrier()

    @pl.when(cid == 0)
    def _():
        def fire(ch, buf):
            base = sid * EPT2 + ch * CHS
            pltpu.sync_copy(dst_hbm.at[pl.ds(base, CHS)], idx_d[buf].at[0])
            pltpu.async_copy(tab_hbm.at[pl.ds(base, CHS)], rows[buf],
                             sem[buf])

        def finish(ch, buf):
            base = sid * EPT2 + ch * CHS
            pltpu.make_async_copy(tab_hbm.at[pl.ds(base, CHS)], rows[buf],
                                  sem[buf]).wait()
            pltpu.sync_copy(rows[buf], acc.at[idx_d[buf].at[0]], add=True)

        fire(0, 0)

        def body(k, _):
            fire(2 * k + 1, 1)
            finish(2 * k, 0)

            @pl.when(k < NCH2S // 2 - 1)
            def _():
                fire(2 * k + 2, 0)

            finish(2 * k + 1, 1)
            return 0

        lax.fori_loop(0, NCH2S // 2, body, 0)

    @pl.when(cid == 1)
    def _():
        _score1_escatter(dst_hbm, e16_hbm, s_out, idx_d1, e16b1, e128, acc,
                         sid)

    plsc.subcore_barrier()

    @pl.when(cid == 0)
    def _():
        pltpu.sync_copy(acc.at[pl.ds(base_r, RPT)],
                        agg_out.at[pl.ds(base_r, RPT)])

    @pl.when(cid == 1)
    def _():
        pltpu.sync_copy(acc.at[pl.ds(base_r, RPT)],
                        s_out.at[pl.ds(base_r, RPT)])


BRV = 512
NBV = V_PAD // BRV
BRE = 32
ER = E_PAD // 128
NBE = ER // BRE


def _full(shape):
    return pl.BlockSpec(shape, lambda i: tuple(0 for _ in shape))


def _rows(bs, *rest):
    return pl.BlockSpec((bs,) + rest, lambda i: (i,) + tuple(0 for _ in rest))


def _node_a_body(hv_ref, pnwt_ref, pnb_ref, w1nt_ref, w2c_ref,
                 hvnew_ref, hvp1_ref, sn_ref):
    x = hv_ref[...]
    hn = x @ pnwt_ref[...] + pnb_ref[...]
    hn = jnp.where(hn > 0, hn, 0.01 * hn)
    hvnew_ref[...] = hn
    hvp1_ref[...] = x @ w1nt_ref[...]
    sn_ref[...] = hn @ w2c_ref[...]


_node_a = pl.pallas_call(
    _node_a_body,
    grid=(NBV,),
    in_specs=[_rows(BRV, D), _full((D, G)), _full((1, G)), _full((D, G)),
              _full((G, G))],
    out_specs=(_rows(BRV, G), _rows(BRV, G), _rows(BRV, G)),
    out_shape=(jax.ShapeDtypeStruct((V_PAD, G), jnp.float32),
               jax.ShapeDtypeStruct((V_PAD, G), jnp.float32),
               jax.ShapeDtypeStruct((V_PAD, G), jnp.float32)),
)


def _efp_body(ef_ref, w_ref, b_ref, out_ref):
    out_ref[...] = ef_ref[...] @ w_ref[...] + b_ref[...]


_efp_k = pl.pallas_call(
    _efp_body,
    grid=(NBE,),
    in_specs=[_rows(BRE * 128, DE), _full((DE, G)), _full((1, G))],
    out_specs=_rows(BRE * 128, G),
    out_shape=jax.ShapeDtypeStruct((E_PAD, G), jnp.float32),
)


def _elogit_body(he1_ref, sdn_ref, w2e_ref, b2_ref, lg_ref, bmax_ref):
    t = jnp.sum(he1_ref[...] * w2e_ref[...][None], axis=2)
    x = sdn_ref[...] + t + b2_ref[...]
    lg = jnp.where(x > 0, x, 0.01 * x)
    lg_ref[...] = lg
    bmax_ref[...] = jnp.max(lg, axis=0, keepdims=True)[None]


_elogit = pl.pallas_call(
    _elogit_body,
    grid=(NBE,),
    in_specs=[_rows(BRE, 128, G), _rows(BRE, 128), _full((1, G)),
              _full((1, 128))],
    out_specs=(_rows(BRE, 128), _rows(1, 1, 128)),
    out_shape=(jax.ShapeDtypeStruct((ER, 128), jnp.float32),
               jax.ShapeDtypeStruct((NBE, 1, 128), jnp.float32)),
)


def _escale_body(lg_ref, m_ref, he1_ref, e_ref, sc_ref):
    e = jnp.exp(lg_ref[...] - m_ref[...])
    e_ref[...] = e
    sc_ref[...] = he1_ref[...] * e[:, :, None]


_escale = pl.pallas_call(
    _escale_body,
    grid=(NBE,),
    in_specs=[_rows(BRE, 128), _full((1, 128)), _rows(BRE, 128, G)],
    out_specs=(_rows(BRE, 128), _rows(BRE, 128, G)),
    out_shape=(jax.ShapeDtypeStruct((ER, 128), jnp.float32),
               jax.ShapeDtypeStruct((ER, 128, G), jnp.float32)),
)


def _elayer_body(d_ref, s_ref, b_ref, m_ref, e_ref):
    x = d_ref[...] + s_ref[...] + b_ref[...]
    lg = jnp.where(x > 0, x, 0.01 * x)
    e_ref[...] = jnp.exp(lg - m_ref[...])


_elayer = pl.pallas_call(
    _elayer_body,
    grid=(NBE,),
    in_specs=[_rows(BRE, 128), _rows(BRE, 128), _full((1, 128)),
              _full((1, 128))],
    out_specs=_rows(BRE, 128),
    out_shape=jax.ShapeDtypeStruct((ER, 128), jnp.float32),
)


def _gru_block(ctx, h, wih_t, whh_t, bih, bhh):
    gi = ctx @ wih_t + bih
    gh = h @ whh_t + bhh
    r = jax.nn.sigmoid(gi[:, :G] + gh[:, :G])
    z = jax.nn.sigmoid(gi[:, G:2 * G] + gh[:, G:2 * G])
    n = jnp.tanh(gi[:, 2 * G:] + r * gh[:, 2 * G:])
    node = (1.0 - z) * n + z * h
    return jnp.maximum(node, 0.0)


def _tables_block(node, packw, pnwt, pnb, scol_ref, hvproj_ref, bmax_ref):
    scol = node @ packw
    scol_ref[...] = scol
    hvproj_ref[...] = node @ pnwt + pnb
    bmax_ref[...] = jnp.max(scol, axis=0, keepdims=True)[None]


def _comb_gc_body(agg0_ref, agg1_ref, s16_ref, hvnew_ref, etwt_ref, etb_ref,
                  wih_ref, whh_ref, bih_ref, bhh_ref,
                  packw_ref, pnwt_ref, pnb_ref,
                  node_ref, scol_ref, hvproj_ref, bmax_ref):
    agg = agg0_ref[...] + agg1_ref[...]
    s = s16_ref[...][:, 0:1]
    denom = s + 1e-9
    c = (agg @ etwt_ref[...]) / denom + (s / denom) * etb_ref[...]
    ctx = jnp.where(c > 0, c, jnp.exp(c) - 1.0)
    node = _gru_block(ctx, hvnew_ref[...], wih_ref[...], whh_ref[...],
                      bih_ref[...], bhh_ref[...])
    node_ref[...] = node
    _tables_block(node, packw_ref[...], pnwt_ref[...], pnb_ref[...],
                  scol_ref, hvproj_ref, bmax_ref)


_comb_gc = pl.pallas_call(
    _comb_gc_body,
    grid=(NBV,),
    in_specs=[_rows(BRV, G), _rows(BRV, G), _rows(BRV, 16), _rows(BRV, G),
              _full((G, G)), _full((1, G)),
              _full((G, 3 * G)), _full((G, 3 * G)), _full((1, 3 * G)),
              _full((1, 3 * G)),
              _full((G, G)), _full((G, G)), _full((1, G))],
    out_specs=(_rows(BRV, G), _rows(BRV, G), _rows(BRV, G),
               _rows(1, 1, 128)),
    out_shape=(jax.ShapeDtypeStruct((V_PAD, G), jnp.float32),
               jax.ShapeDtypeStruct((V_PAD, G), jnp.float32),
               jax.ShapeDtypeStruct((V_PAD, G), jnp.float32),
               jax.ShapeDtypeStruct((NBV, 1, 128), jnp.float32)),
)


def _comb_layer_body(agg0_ref, agg1_ref, s16_ref, h_ref,
                     wih_ref, whh_ref, bih_ref, bhh_ref,
                     packw_ref, pnwt_ref, pnb_ref,
                     node_ref, scol_ref, hvproj_ref, bmax_ref):
    agg = agg0_ref[...] + agg1_ref[...]
    s = s16_ref[...][:, 0:1]
    c = agg / (s + 1e-9)
    ctx = jnp.where(c > 0, c, jnp.exp(c) - 1.0)
    node = _gru_block(ctx, h_ref[...], wih_ref[...], whh_ref[...],
                      bih_ref[...], bhh_ref[...])
    node_ref[...] = node
    _tables_block(node, packw_ref[...], pnwt_ref[...], pnb_ref[...],
                  scol_ref, hvproj_ref, bmax_ref)


_comb_layer = pl.pallas_call(
    _comb_layer_body,
    grid=(NBV,),
    in_specs=[_rows(BRV, G), _rows(BRV, G), _rows(BRV, 16), _rows(BRV, G),
              _full((G, 3 * G)), _full((G, 3 * G)), _full((1, 3 * G)),
              _full((1, 3 * G)),
              _full((G, G)), _full((G, G)), _full((1, G))],
    out_specs=(_rows(BRV, G), _rows(BRV, G), _rows(BRV, G),
               _rows(1, 1, 128)),
    out_shape=(jax.ShapeDtypeStruct((V_PAD, G), jnp.float32),
               jax.ShapeDtypeStruct((V_PAD, G), jnp.float32),
               jax.ShapeDtypeStruct((V_PAD, G), jnp.float32),
               jax.ShapeDtypeStruct((NBV, 1, 128), jnp.float32)),
)


def _leaky(x):
    return jax.nn.leaky_relu(x, negative_slope=0.01)


def _gru_update(x, h, W_ih, W_hh, b_ih, b_hh):
    gi = x @ W_ih.T + b_ih
    gh = h @ W_hh.T + b_hh
    i_r, i_z, i_n = jnp.split(gi, 3, axis=1)
    h_r, h_z, h_n = jnp.split(gh, 3, axis=1)
    r = jax.nn.sigmoid(i_r + h_r)
    z = jax.nn.sigmoid(i_z + h_z)
    n = jnp.tanh(i_n + r * h_n)
    return (1.0 - z) * n + z * h


def kernel(node_feats, edge_feats, edge_index,
           gc_pn_W, gc_pn_b, gc_pe1_W, gc_pe1_b, gc_pe2_W, gc_pe2_b,
           gc_et_W, gc_et_b, gc_gru_Wih, gc_gru_Whh, gc_gru_bih, gc_gru_bhh,
           l0_pe_W, l0_pe_b, l0_pn_W, l0_pn_b,
           l0_gru_Wih, l0_gru_Whh, l0_gru_bih, l0_gru_bhh,
           l1_pe_W, l1_pe_b, l1_pn_W, l1_pn_b,
           l1_gru_Wih, l1_gru_Whh, l1_gru_bih, l1_gru_bhh,
           pred_W, pred_b):
    f32 = jnp.float32
    src = jnp.full((E_PAD,), DUMMY, jnp.int32).at[:E].set(
        edge_index[0].astype(jnp.int32))
    dst = jnp.full((E_PAD,), DUMMY, jnp.int32).at[:E].set(
        edge_index[1].astype(jnp.int32))
    hv_pad = jnp.zeros((V_PAD, D), f32).at[:V].set(node_feats)
    ef_pad = jnp.zeros((E_PAD, DE), f32).at[:E].set(edge_feats)

    def col_mat(*cols):
        w = jnp.zeros((G, G), f32)
        for k, c in enumerate(cols):
            w = w.at[:, k].set(c)
        return w

    hv_new, hv_p1_pad, sn_mat = _node_a(
        hv_pad, gc_pn_W.T, gc_pn_b[None], gc_pe1_W[:, :D].T,
        col_mat(gc_pe2_W[0, :G]))
    sn_pad = sn_mat[:, 0]
    efp_pad = _efp_k(ef_pad, gc_pe1_W[:, D:].T, gc_pe1_b[None])

    he1_pad, sdn = _g1_kernel(src, dst, hv_p1_pad, sn_pad, efp_pad)

    he1_3d = he1_pad.reshape(ER, 128, G)
    lg2, bmax = _elogit(he1_3d, sdn.reshape(ER, 128),
                        gc_pe2_W[0:1, G:], jnp.full((1, 128), gc_pe2_b[0]))
    M = jnp.max(bmax)
    e2, she1 = _escale(lg2, jnp.full((1, 128), M), he1_3d)
    e16 = jnp.broadcast_to(e2.reshape(E_PAD)[:, None], (E_PAD, 16))

    agg2c = _scatseq_kernel(dst, she1.reshape(E_PAD, G))
    s2c = _sscat_kernel(dst, e16)
    s16sum = (s2c[0] + s2c[1])[:, :16]

    node, scol, hvproj, bmax = _comb_gc(
        agg2c[0], agg2c[1], s16sum, hv_new, gc_et_W.T, gc_et_b[None],
        gc_gru_Wih.T, gc_gru_Whh.T, gc_gru_bih[None], gc_gru_bhh[None],
        col_mat(l0_pe_W[0, :G], l0_pe_W[0, G:]), l0_pn_W.T, l0_pn_b[None])

    layer_w = (
        (l0_pe_b, l0_gru_Wih, l0_gru_Whh, l0_gru_bih, l0_gru_bhh,
         col_mat(l1_pe_W[0, :G], l1_pe_W[0, G:]), l1_pn_W.T, l1_pn_b[None]),
        (l1_pe_b, l1_gru_Wih, l1_gru_Whh, l1_gru_bih, l1_gru_bhh,
         col_mat(pred_W[0]), jnp.zeros((G, G), f32), jnp.zeros((1, G), f32)),
    )
    for (pe_b, Wih, Whh, bih, bhh, next_packw, next_pnwt, next_pnb) in layer_w:
        b = pe_b[0]
        Mub = _leaky(jnp.max(bmax[:, 0, 0]) + jnp.max(bmax[:, 0, 1]) + b)
        dsc_pad = scol[:, 0]
        ssc_pad = scol[:, 1]

        d_e, s_e = _gat2_kernel(src, dst, dsc_pad, ssc_pad)
        e2 = _elayer(d_e.reshape(ER, 128), s_e.reshape(ER, 128),
                     jnp.full((1, 128), b), jnp.full((1, 128), Mub))
        e16 = jnp.broadcast_to(e2.reshape(E_PAD)[:, None], (E_PAD, 16))

        agg2c = _scat_kernel(src, dst, hvproj, e16)
        s2c = _sscat_kernel(dst, e16)
        s16sum = (s2c[0] + s2c[1])[:, :16]
        node, scol, hvproj, bmax = _comb_layer(
            agg2c[0], agg2c[1], s16sum, node,
            Wih.T, Whh.T, bih[None], bhh[None],
            next_packw, next_pnwt, next_pnb)

    return scol[:V, 0:1] + pred_b


# speedup vs baseline: 5.4048x; 1.0237x over previous
import functools

import jax
import jax.numpy as jnp
from jax import lax
from jax.experimental import pallas as pl
from jax.experimental.pallas import tpu as pltpu
from jax.experimental.pallas import tpu_sc as plsc

V = 10000
E = 320000
D = 128
DE = 16
G = 128

NC = 2
NS = 16
NW = NC * NS
V_PAD = 10240
E_PAD = 327680
EPW = E_PAD // NW
CH = 128
NCHUNK = EPW // CH
CHS = 64
NCHS = EPW // CHS
RPT = V_PAD // NS
DUMMY = V_PAD - 1

_mesh = plsc.VectorSubcoreMesh(core_axis_name="c", subcore_axis_name="s")


def _leaky_v(x):
    return jnp.where(x > 0, x, x * 0.01)


def _wid():
    return lax.axis_index("s") * NC + lax.axis_index("c")


@functools.partial(
    pl.kernel,
    out_type=(
        jax.ShapeDtypeStruct((E_PAD, G), jnp.float32),
        jax.ShapeDtypeStruct((E_PAD,), jnp.float32),
    ),
    mesh=_mesh,
    scratch_types=[
        [pltpu.VMEM((CH,), jnp.int32)] * 2,
        [pltpu.VMEM((CH,), jnp.int32)] * 2,
        [pltpu.VMEM((CH, G), jnp.float32)] * 2,
        [pltpu.VMEM((CH, G), jnp.float32)] * 2,
        [pltpu.VMEM((CH,), jnp.float32)] * 2,
        [pltpu.SemaphoreType.DMA] * 2,
    ],
)
def _g1_kernel(src_hbm, dst_hbm, hvp1_hbm, sn_hbm, efp_hbm,
               he1_hbm, sdn_hbm,
               idx_s, idx_d, rows, efb, dnv, sem):
    wid = _wid()

    def fire(ch, buf):
        base = wid * EPW + ch * CH
        pltpu.sync_copy(src_hbm.at[pl.ds(base, CH)], idx_s[buf])
        pltpu.sync_copy(dst_hbm.at[pl.ds(base, CH)], idx_d[buf])
        pltpu.async_copy(efp_hbm.at[pl.ds(base, CH)], efb[buf], sem[buf])
        pltpu.async_copy(hvp1_hbm.at[idx_s[buf]], rows[buf], sem[buf])
        pltpu.async_copy(sn_hbm.at[idx_d[buf]], dnv[buf], sem[buf])

    def finish(ch, buf):
        base = wid * EPW + ch * CH
        pltpu.make_async_copy(efp_hbm.at[pl.ds(base, CH)], efb[buf],
                              sem[buf]).wait()
        pltpu.make_async_copy(hvp1_hbm.at[idx_s[buf]], rows[buf],
                              sem[buf]).wait()
        pltpu.make_async_copy(sn_hbm.at[idx_d[buf]], dnv[buf],
                              sem[buf]).wait()
        r = rows[buf]
        ef = efb[buf]

        def edge(g, _):
            for il in range(4):
                i = g * 4 + il
                for j in range(G // 16):
                    u = r[i, pl.ds(j * 16, 16)] + ef[i, pl.ds(j * 16, 16)]
                    r[i, pl.ds(j * 16, 16)] = _leaky_v(u)
            return 0

        lax.fori_loop(0, CH // 4, edge, 0)
        pltpu.sync_copy(r, he1_hbm.at[pl.ds(base, CH)])
        pltpu.sync_copy(dnv[buf], sdn_hbm.at[pl.ds(base, CH)])

    fire(0, 0)

    def body(k, _):
        fire(2 * k + 1, 1)
        finish(2 * k, 0)

        @pl.when(k < NCHUNK // 2 - 1)
        def _():
            fire(2 * k + 2, 0)

        finish(2 * k + 1, 1)
        return 0

    lax.fori_loop(0, NCHUNK // 2, body, 0)


@functools.partial(
    pl.kernel,
    out_type=(
        jax.ShapeDtypeStruct((E_PAD,), jnp.float32),
        jax.ShapeDtypeStruct((E_PAD,), jnp.float32),
    ),
    mesh=_mesh,
    scratch_types=[
        [pltpu.VMEM((CH,), jnp.int32)] * 2,
        [pltpu.VMEM((CH,), jnp.int32)] * 2,
        [pltpu.VMEM((CH,), jnp.float32)] * 2,
        [pltpu.VMEM((CH,), jnp.float32)] * 2,
        [pltpu.SemaphoreType.DMA] * 2,
    ],
)
def _gat2_kernel(src_hbm, dst_hbm, dsc_hbm, ssc_hbm,
                 de_hbm, se_hbm,
                 idx_s, idx_d, dval, sval, sem):
    wid = _wid()

    def fire(ch, buf):
        base = wid * EPW + ch * CH
        pltpu.sync_copy(src_hbm.at[pl.ds(base, CH)], idx_s[buf])
        pltpu.sync_copy(dst_hbm.at[pl.ds(base, CH)], idx_d[buf])
        pltpu.async_copy(dsc_hbm.at[idx_d[buf]], dval[buf], sem[buf])
        pltpu.async_copy(ssc_hbm.at[idx_s[buf]], sval[buf], sem[buf])

    def finish(ch, buf):
        base = wid * EPW + ch * CH
        pltpu.make_async_copy(dsc_hbm.at[idx_d[buf]], dval[buf],
                              sem[buf]).wait()
        pltpu.make_async_copy(ssc_hbm.at[idx_s[buf]], sval[buf],
                              sem[buf]).wait()
        pltpu.sync_copy(dval[buf], de_hbm.at[pl.ds(base, CH)])
        pltpu.sync_copy(sval[buf], se_hbm.at[pl.ds(base, CH)])

    fire(0, 0)

    def body(k, _):
        fire(2 * k + 1, 1)
        finish(2 * k, 0)
        fire(2 * k + 2, 0)
        finish(2 * k + 1, 1)
        return 0

    lax.fori_loop(0, NCHUNK // 2 - 1, body, 0)
    fire(NCHUNK - 1, 1)
    finish(NCHUNK - 2, 0)
    finish(NCHUNK - 1, 1)


@functools.partial(
    pl.kernel,
    out_type=jax.ShapeDtypeStruct((NC, V_PAD, G), jnp.float32),
    mesh=_mesh,
    scratch_types=[
        [pltpu.VMEM((CH,), jnp.int32)] * 1,
        [pltpu.VMEM((1, CH), jnp.int32)] * 1,
        [pltpu.VMEM((CH, G), jnp.float32)] * 1,
        [pltpu.VMEM((CH, 16), jnp.float32)] * 1,
        pltpu.VMEM((16, G), jnp.float32),
        pltpu.VMEM_SHARED((V_PAD, G), jnp.float32),
        [pltpu.SemaphoreType.DMA] * 2,
    ],
)
def _scat_kernel(rid_hbm, dst_hbm, tab_hbm, e16_hbm,
                 agg_out,
                 idx_s, idx_d, rows, e16b, zrow, agg_acc, sem):
    cid = lax.axis_index("c")
    sid = lax.axis_index("s")
    wid = sid * NC + cid

    zf = jnp.zeros((16,), jnp.float32)
    for i in range(16):
        for j in range(G // 16):
            zrow[i, pl.ds(j * 16, 16)] = zf
    base_r = sid * RPT

    def zb(t, _):
        pltpu.sync_copy(zrow, agg_acc.at[pl.ds(base_r + t * 16, 16)])
        return 0

    lax.fori_loop(0, RPT // 16, zb, 0)
    plsc.subcore_barrier()

    def chunk(ch, _):
        base = wid * EPW + ch * CH
        pltpu.async_copy(rid_hbm.at[pl.ds(base, CH)], idx_s[0], sem[0])
        pltpu.async_copy(e16_hbm.at[pl.ds(base, CH)], e16b[0], sem[0])

        @pl.when(ch > 0)
        def _():
            pltpu.make_async_copy(rows[0], agg_acc.at[idx_d[0].at[0]],
                                  sem[1]).wait()

        pltpu.async_copy(dst_hbm.at[pl.ds(base, CH)], idx_d[0].at[0], sem[0])
        pltpu.make_async_copy(rid_hbm.at[pl.ds(base, CH)], idx_s[0],
                              sem[0]).wait()
        pltpu.make_async_copy(dst_hbm.at[pl.ds(base, CH)], idx_d[0].at[0],
                              sem[0]).wait()
        pltpu.make_async_copy(e16_hbm.at[pl.ds(base, CH)], e16b[0],
                              sem[0]).wait()
        pltpu.async_copy(tab_hbm.at[idx_s[0]], rows[0], sem[0]).wait()
        r = rows[0]
        ev = e16b[0]

        def edge(g, _):
            for il in range(4):
                i = g * 4 + il
                eb = ev[i]
                for j in range(G // 16):
                    r[i, pl.ds(j * 16, 16)] = r[i, pl.ds(j * 16, 16)] * eb
            return 0

        lax.fori_loop(0, CH // 4, edge, 0)
        pltpu.async_copy(r, agg_acc.at[idx_d[0].at[0]], sem[1], add=True)
        return 0

    lax.fori_loop(0, NCHUNK, chunk, 0)
    pltpu.make_async_copy(rows[0], agg_acc.at[idx_d[0].at[0]], sem[1]).wait()
    plsc.subcore_barrier()
    pltpu.sync_copy(agg_acc.at[pl.ds(base_r, RPT)],
                    agg_out.at[cid, pl.ds(base_r, RPT)])


@functools.partial(
    pl.kernel,
    out_type=jax.ShapeDtypeStruct((NC, V_PAD, G), jnp.float32),
    mesh=_mesh,
    scratch_types=[
        [pltpu.VMEM((1, CH), jnp.int32)] * 1,
        [pltpu.VMEM((CH, G), jnp.float32)] * 1,
        [pltpu.VMEM((CH, 16), jnp.float32)] * 1,
        pltpu.VMEM((16, G), jnp.float32),
        pltpu.VMEM_SHARED((V_PAD, G), jnp.float32),
        [pltpu.SemaphoreType.DMA] * 1,
    ],
)
def _scatseq_kernel(dst_hbm, tab_hbm,
                    agg_out,
                    idx_d, rows, e16b, zrow, agg_acc, sem):
    cid = lax.axis_index("c")
    sid = lax.axis_index("s")
    wid = sid * NC + cid

    zf = jnp.zeros((16,), jnp.float32)
    for i in range(16):
        for j in range(G // 16):
            zrow[i, pl.ds(j * 16, 16)] = zf
    base_r = sid * RPT

    def zb(t, _):
        pltpu.sync_copy(zrow, agg_acc.at[pl.ds(base_r + t * 16, 16)])
        return 0

    lax.fori_loop(0, RPT // 16, zb, 0)
    plsc.subcore_barrier()

    def chunk(ch, _):
        base = wid * EPW + ch * CH
        pltpu.async_copy(dst_hbm.at[pl.ds(base, CH)], idx_d[0].at[0], sem[0])
        pltpu.async_copy(tab_hbm.at[pl.ds(base, CH)], rows[0], sem[0])
        pltpu.make_async_copy(dst_hbm.at[pl.ds(base, CH)], idx_d[0].at[0],
                              sem[0]).wait()
        pltpu.make_async_copy(tab_hbm.at[pl.ds(base, CH)], rows[0],
                              sem[0]).wait()
        pltpu.sync_copy(rows[0], agg_acc.at[idx_d[0].at[0]], add=True)
        return 0

    lax.fori_loop(0, NCHUNK, chunk, 0)
    plsc.subcore_barrier()
    pltpu.sync_copy(agg_acc.at[pl.ds(base_r, RPT)],
                    agg_out.at[cid, pl.ds(base_r, RPT)])


@functools.partial(
    pl.kernel,
    out_type=jax.ShapeDtypeStruct((NC, V_PAD, G), jnp.float32),
    mesh=_mesh,
    scratch_types=[
        pltpu.VMEM((1, CH), jnp.int32),
        pltpu.VMEM((CH, 16), jnp.float32),
        pltpu.VMEM((CH, G), jnp.float32),
        pltpu.VMEM((16, G), jnp.float32),
        pltpu.VMEM_SHARED((V_PAD, G), jnp.float32),
        pltpu.SemaphoreType.DMA,
    ],
)
def _sscat_kernel(dst_hbm, e16_hbm,
                  s_out,
                  idx_d, e16b, e128, zrow, s_acc, sem):
    cid = lax.axis_index("c")
    sid = lax.axis_index("s")
    wid = sid * NC + cid

    zf = jnp.zeros((16,), jnp.float32)
    for i in range(16):
        for j in range(G // 16):
            zrow[i, pl.ds(j * 16, 16)] = zf
    base_r = sid * RPT

    def zb(t, _):
        pltpu.sync_copy(zrow, s_acc.at[pl.ds(base_r + t * 16, 16)])
        return 0

    lax.fori_loop(0, RPT // 16, zb, 0)
    plsc.subcore_barrier()

    def chunk(ch, _):
        base = wid * EPW + ch * CH
        pltpu.async_copy(dst_hbm.at[pl.ds(base, CH)], idx_d.at[0], sem)
        pltpu.async_copy(e16_hbm.at[pl.ds(base, CH)], e16b, sem)
        pltpu.make_async_copy(dst_hbm.at[pl.ds(base, CH)], idx_d.at[0],
                              sem).wait()
        pltpu.make_async_copy(e16_hbm.at[pl.ds(base, CH)], e16b, sem).wait()

        def edge(g, _):
            for il in range(4):
                i = g * 4 + il
                eb = e16b[i]
                for j in range(G // 16):
                    e128[i, pl.ds(j * 16, 16)] = eb
            return 0

        lax.fori_loop(0, CH // 4, edge, 0)
        pltpu.sync_copy(e128, s_acc.at[idx_d.at[0]], add=True)
        return 0

    lax.fori_loop(0, NCHUNK, chunk, 0)
    plsc.subcore_barrier()
    pltpu.sync_copy(s_acc.at[pl.ds(base_r, RPT)],
                    s_out.at[cid, pl.ds(base_r, RPT)])


EPT2 = E_PAD // NS
NCH2S = EPT2 // CHS
NCH2 = EPT2 // CH


def _zero_acc(zrow, acc, sid):
    zf = jnp.zeros((16,), jnp.float32)
    for i in range(16):
        for j in range(G // 16):
            zrow[i, pl.ds(j * 16, 16)] = zf
    base_r = sid * RPT

    def zb(t, _):
        pltpu.sync_copy(zrow, acc.at[pl.ds(base_r + t * 16, 16)])
        return 0

    lax.fori_loop(0, RPT // 16, zb, 0)
    return base_r


CH1 = 32


def _score1_escatter(dst_hbm, e16_hbm, s_out, idx_d, e16b, e128, acc, sid):
    def chunk(ch, _):
        base = sid * EPT2 + ch * CH1
        pltpu.sync_copy(dst_hbm.at[pl.ds(base, CH1)], idx_d.at[0])
        pltpu.sync_copy(e16_hbm.at[pl.ds(base, CH1)], e16b)

        def edge(g, _):
            for il in range(4):
                i = g * 4 + il
                eb = e16b[i]
                for j in range(G // 16):
                    e128[i, pl.ds(j * 16, 16)] = eb
            return 0

        lax.fori_loop(0, CH1 // 4, edge, 0)
        pltpu.sync_copy(e128, acc.at[idx_d.at[0]], add=True)
        return 0

    lax.fori_loop(0, EPT2 // CH1, chunk, 0)


@functools.partial(
    pl.kernel,
    out_type=(
        jax.ShapeDtypeStruct((V_PAD, G), jnp.float32),
        jax.ShapeDtypeStruct((V_PAD, G), jnp.float32),
    ),
    mesh=_mesh,
    scratch_types=[
        [pltpu.VMEM((CHS,), jnp.int32)] * 2,
        [pltpu.VMEM((1, CHS), jnp.int32)] * 2,
        [pltpu.VMEM((CHS, G), jnp.float32)] * 2,
        [pltpu.VMEM((CHS, 16), jnp.float32)] * 2,
        pltpu.VMEM((1, 32), jnp.int32),
        pltpu.VMEM((32, 16), jnp.float32),
        pltpu.VMEM((32, G), jnp.float32),
        pltpu.VMEM((16, G), jnp.float32),
        pltpu.VMEM_SHARED((V_PAD, G), jnp.float32),
        [pltpu.SemaphoreType.DMA] * 2,
    ],
)
def _dualg_kernel(src_hbm, dst_hbm, tab_hbm, e16_hbm,
                  agg_out, s_out,
                  idx_s, idx_d, rows, e16b, idx_d1, e16b1, e128, zrow, acc,
                  sem):
    cid = lax.axis_index("c")
    sid = lax.axis_index("s")
    base_r = _zero_acc(zrow, acc, sid)
    plsc.subcore_barrier()

    @pl.when(cid == 0)
    def _():
        def fire(ch, buf):
            base = sid * EPT2 + ch * CHS
            pltpu.sync_copy(src_hbm.at[pl.ds(base, CHS)], idx_s[buf])
            pltpu.sync_copy(dst_hbm.at[pl.ds(base, CHS)], idx_d[buf].at[0])
            pltpu.sync_copy(e16_hbm.at[pl.ds(base, CHS)], e16b[buf])
            pltpu.async_copy(tab_hbm.at[idx_s[buf]], rows[buf], sem[buf])

        def finish(ch, buf):
            pltpu.make_async_copy(tab_hbm.at[idx_s[buf]], rows[buf],
                                  sem[buf]).wait()
            r = rows[buf]
            ev = e16b[buf]

            def edge(g, _):
                for il in range(4):
                    i = g * 4 + il
                    eb = ev[i]
                    for j in range(G // 16):
                        r[i, pl.ds(j * 16, 16)] = r[i, pl.ds(j * 16, 16)] * eb
                return 0

            lax.fori_loop(0, CHS // 4, edge, 0)
            pltpu.sync_copy(r, acc.at[idx_d[buf].at[0]], add=True)

        fire(0, 0)

        def body(k, _):
            fire(2 * k + 1, 1)
            finish(2 * k, 0)

            @pl.when(k < NCH2S // 2 - 1)
            def _():
                fire(2 * k + 2, 0)

            finish(2 * k + 1, 1)
            return 0

        lax.fori_loop(0, NCH2S // 2, body, 0)

    @pl.when(cid == 1)
    def _():
        _score1_escatter(dst_hbm, e16_hbm, s_out, idx_d1, e16b1, e128, acc,
                         sid)

    plsc.subcore_barrier()

    @pl.when(cid == 0)
    def _():
        pltpu.sync_copy(acc.at[pl.ds(base_r, RPT)],
                        agg_out.at[pl.ds(base_r, RPT)])

    @pl.when(cid == 1)
    def _():
        pltpu.sync_copy(acc.at[pl.ds(base_r, RPT)],
                        s_out.at[pl.ds(base_r, RPT)])


@functools.partial(
    pl.kernel,
    out_type=(
        jax.ShapeDtypeStruct((V_PAD, G), jnp.float32),
        jax.ShapeDtypeStruct((V_PAD, G), jnp.float32),
    ),
    mesh=_mesh,
    scratch_types=[
        [pltpu.VMEM((1, CHS), jnp.int32)] * 2,
        [pltpu.VMEM((CHS, G), jnp.float32)] * 2,
        pltpu.VMEM((1, 32), jnp.int32),
        pltpu.VMEM((32, 16), jnp.float32),
        pltpu.VMEM((32, G), jnp.float32),
        pltpu.VMEM((16, G), jnp.float32),
        pltpu.VMEM_SHARED((V_PAD, G), jnp.float32),
        [pltpu.SemaphoreType.DMA] * 2,
    ],
)
def _dualseq_kernel(dst_hbm, tab_hbm, e16_hbm,
                    agg_out, s_out,
                    idx_d, rows, idx_d1, e16b1, e128, zrow, acc, sem):
    cid = lax.axis_index("c")
    sid = lax.axis_index("s")
    base_r = _zero_acc(zrow, acc, sid)
    plsc.subcore_barrier()

    @pl.when(cid == 0)
    def _():
        def fire(ch, buf):
            base = sid * EPT2 + ch * CHS
            pltpu.sync_copy(dst_hbm.at[pl.ds(base, CHS)], idx_d[buf].at[0])
            pltpu.async_copy(tab_hbm.at[pl.ds(base, CHS)], rows[buf],
                             sem[buf])

        def finish(ch, buf):
            base = sid * EPT2 + ch * CHS
            pltpu.make_async_copy(tab_hbm.at[pl.ds(base, CHS)], rows[buf],
                                  sem[buf]).wait()
            pltpu.sync_copy(rows[buf], acc.at[idx_d[buf].at[0]], add=True)

        fire(0, 0)

        def body(k, _):
            fire(2 * k + 1, 1)
            finish(2 * k, 0)

            @pl.when(k < NCH2S // 2 - 1)
            def _():
                fire(2 * k + 2, 0)

            finish(2 * k + 1, 1)
            return 0

        lax.fori_loop(0, NCH2S // 2, body, 0)

    @pl.when(cid == 1)
    def _():
        _score1_escatter(dst_hbm, e16_hbm, s_out, idx_d1, e16b1, e128, acc,
                         sid)

    plsc.subcore_barrier()

    @pl.when(cid == 0)
    def _():
        pltpu.sync_copy(acc.at[pl.ds(base_r, RPT)],
                        agg_out.at[pl.ds(base_r, RPT)])

    @pl.when(cid == 1)
    def _():
        pltpu.sync_copy(acc.at[pl.ds(base_r, RPT)],
                        s_out.at[pl.ds(base_r, RPT)])


BRV = 512
NBV = V_PAD // BRV
BRE = 32
ER = E_PAD // 128
NBE = ER // BRE


def _full(shape):
    return pl.BlockSpec(shape, lambda i: tuple(0 for _ in shape))


def _rows(bs, *rest):
    return pl.BlockSpec((bs,) + rest, lambda i: (i,) + tuple(0 for _ in rest))


def _node_a_body(hv_ref, pnwt_ref, pnb_ref, w1nt_ref, w2c_ref,
                 hvnew_ref, hvp1_ref, sn_ref):
    x = hv_ref[...]
    hn = x @ pnwt_ref[...] + pnb_ref[...]
    hn = jnp.where(hn > 0, hn, 0.01 * hn)
    hvnew_ref[...] = hn
    hvp1_ref[...] = x @ w1nt_ref[...]
    sn_ref[...] = hn @ w2c_ref[...]


_node_a = pl.pallas_call(
    _node_a_body,
    grid=(NBV,),
    in_specs=[_rows(BRV, D), _full((D, G)), _full((1, G)), _full((D, G)),
              _full((G, G))],
    out_specs=(_rows(BRV, G), _rows(BRV, G), _rows(BRV, G)),
    out_shape=(jax.ShapeDtypeStruct((V_PAD, G), jnp.float32),
               jax.ShapeDtypeStruct((V_PAD, G), jnp.float32),
               jax.ShapeDtypeStruct((V_PAD, G), jnp.float32)),
)


def _efp_body(ef_ref, w_ref, b_ref, out_ref):
    out_ref[...] = ef_ref[...] @ w_ref[...] + b_ref[...]


_efp_k = pl.pallas_call(
    _efp_body,
    grid=(NBE,),
    in_specs=[_rows(BRE * 128, DE), _full((DE, G)), _full((1, G))],
    out_specs=_rows(BRE * 128, G),
    out_shape=jax.ShapeDtypeStruct((E_PAD, G), jnp.float32),
)


def _elogit_body(he1_ref, sdn_ref, w2e_ref, b2_ref, lg_ref, bmax_ref):
    t = jnp.sum(he1_ref[...] * w2e_ref[...][None], axis=2)
    x = sdn_ref[...] + t + b2_ref[...]
    lg = jnp.where(x > 0, x, 0.01 * x)
    lg_ref[...] = lg
    bmax_ref[...] = jnp.max(lg, axis=0, keepdims=True)[None]


_elogit = pl.pallas_call(
    _elogit_body,
    grid=(NBE,),
    in_specs=[_rows(BRE, 128, G), _rows(BRE, 128), _full((1, G)),
              _full((1, 128))],
    out_specs=(_rows(BRE, 128), _rows(1, 1, 128)),
    out_shape=(jax.ShapeDtypeStruct((ER, 128), jnp.float32),
               jax.ShapeDtypeStruct((NBE, 1, 128), jnp.float32)),
)


def _escale_body(lg_ref, m_ref, he1_ref, e_ref, sc_ref):
    e = jnp.exp(lg_ref[...] - m_ref[...])
    e_ref[...] = e
    sc_ref[...] = he1_ref[...] * e[:, :, None]


_escale = pl.pallas_call(
    _escale_body,
    grid=(NBE,),
    in_specs=[_rows(BRE, 128), _full((1, 128)), _rows(BRE, 128, G)],
    out_specs=(_rows(BRE, 128), _rows(BRE, 128, G)),
    out_shape=(jax.ShapeDtypeStruct((ER, 128), jnp.float32),
               jax.ShapeDtypeStruct((ER, 128, G), jnp.float32)),
)


def _elayer_body(d_ref, s_ref, b_ref, m_ref, e_ref):
    x = d_ref[...] + s_ref[...] + b_ref[...]
    lg = jnp.where(x > 0, x, 0.01 * x)
    e_ref[...] = jnp.exp(lg - m_ref[...])


_elayer = pl.pallas_call(
    _elayer_body,
    grid=(NBE,),
    in_specs=[_rows(BRE, 128), _rows(BRE, 128), _full((1, 128)),
              _full((1, 128))],
    out_specs=_rows(BRE, 128),
    out_shape=jax.ShapeDtypeStruct((ER, 128), jnp.float32),
)


def _gru_block(ctx, h, wih_t, whh_t, bih, bhh):
    gi = ctx @ wih_t + bih
    gh = h @ whh_t + bhh
    r = jax.nn.sigmoid(gi[:, :G] + gh[:, :G])
    z = jax.nn.sigmoid(gi[:, G:2 * G] + gh[:, G:2 * G])
    n = jnp.tanh(gi[:, 2 * G:] + r * gh[:, 2 * G:])
    node = (1.0 - z) * n + z * h
    return jnp.maximum(node, 0.0)


def _tables_block(node, packw, pnwt, pnb, scol_ref, hvproj_ref, bmax_ref):
    scol = node @ packw
    scol_ref[...] = scol
    hvproj_ref[...] = node @ pnwt + pnb
    bmax_ref[...] = jnp.max(scol, axis=0, keepdims=True)[None]


def _comb_gc_body(agg0_ref, agg1_ref, s16_ref, hvnew_ref, etwt_ref, etb_ref,
                  wih_ref, whh_ref, bih_ref, bhh_ref,
                  packw_ref, pnwt_ref, pnb_ref,
                  node_ref, scol_ref, hvproj_ref, bmax_ref):
    agg = agg0_ref[...] + agg1_ref[...]
    s = s16_ref[...][:, 0:1]
    denom = s + 1e-9
    c = (agg @ etwt_ref[...]) / denom + (s / denom) * etb_ref[...]
    ctx = jnp.where(c > 0, c, jnp.exp(c) - 1.0)
    node = _gru_block(ctx, hvnew_ref[...], wih_ref[...], whh_ref[...],
                      bih_ref[...], bhh_ref[...])
    node_ref[...] = node
    _tables_block(node, packw_ref[...], pnwt_ref[...], pnb_ref[...],
                  scol_ref, hvproj_ref, bmax_ref)


_comb_gc = pl.pallas_call(
    _comb_gc_body,
    grid=(NBV,),
    in_specs=[_rows(BRV, G), _rows(BRV, G), _rows(BRV, 16), _rows(BRV, G),
              _full((G, G)), _full((1, G)),
              _full((G, 3 * G)), _full((G, 3 * G)), _full((1, 3 * G)),
              _full((1, 3 * G)),
              _full((G, G)), _full((G, G)), _full((1, G))],
    out_specs=(_rows(BRV, G), _rows(BRV, G), _rows(BRV, G),
               _rows(1, 1, 128)),
    out_shape=(jax.ShapeDtypeStruct((V_PAD, G), jnp.float32),
               jax.ShapeDtypeStruct((V_PAD, G), jnp.float32),
               jax.ShapeDtypeStruct((V_PAD, G), jnp.float32),
               jax.ShapeDtypeStruct((NBV, 1, 128), jnp.float32)),
)


def _comb_layer_body(agg0_ref, agg1_ref, s16_ref, h_ref,
                     wih_ref, whh_ref, bih_ref, bhh_ref,
                     packw_ref, pnwt_ref, pnb_ref,
                     node_ref, scol_ref, hvproj_ref, bmax_ref):
    agg = agg0_ref[...] + agg1_ref[...]
    s = s16_ref[...][:, 0:1]
    c = agg / (s + 1e-9)
    ctx = jnp.where(c > 0, c, jnp.exp(c) - 1.0)
    node = _gru_block(ctx, h_ref[...], wih_ref[...], whh_ref[...],
                      bih_ref[...], bhh_ref[...])
    node_ref[...] = node
    _tables_block(node, packw_ref[...], pnwt_ref[...], pnb_ref[...],
                  scol_ref, hvproj_ref, bmax_ref)


_comb_layer = pl.pallas_call(
    _comb_layer_body,
    grid=(NBV,),
    in_specs=[_rows(BRV, G), _rows(BRV, G), _rows(BRV, 16), _rows(BRV, G),
              _full((G, 3 * G)), _full((G, 3 * G)), _full((1, 3 * G)),
              _full((1, 3 * G)),
              _full((G, G)), _full((G, G)), _full((1, G))],
    out_specs=(_rows(BRV, G), _rows(BRV, G), _rows(BRV, G),
               _rows(1, 1, 128)),
    out_shape=(jax.ShapeDtypeStruct((V_PAD, G), jnp.float32),
               jax.ShapeDtypeStruct((V_PAD, G), jnp.float32),
               jax.ShapeDtypeStruct((V_PAD, G), jnp.float32),
               jax.ShapeDtypeStruct((NBV, 1, 128), jnp.float32)),
)


def _leaky(x):
    return jax.nn.leaky_relu(x, negative_slope=0.01)


def _gru_update(x, h, W_ih, W_hh, b_ih, b_hh):
    gi = x @ W_ih.T + b_ih
    gh = h @ W_hh.T + b_hh
    i_r, i_z, i_n = jnp.split(gi, 3, axis=1)
    h_r, h_z, h_n = jnp.split(gh, 3, axis=1)
    r = jax.nn.sigmoid(i_r + h_r)
    z = jax.nn.sigmoid(i_z + h_z)
    n = jnp.tanh(i_n + r * h_n)
    return (1.0 - z) * n + z * h


def kernel(node_feats, edge_feats, edge_index,
           gc_pn_W, gc_pn_b, gc_pe1_W, gc_pe1_b, gc_pe2_W, gc_pe2_b,
           gc_et_W, gc_et_b, gc_gru_Wih, gc_gru_Whh, gc_gru_bih, gc_gru_bhh,
           l0_pe_W, l0_pe_b, l0_pn_W, l0_pn_b,
           l0_gru_Wih, l0_gru_Whh, l0_gru_bih, l0_gru_bhh,
           l1_pe_W, l1_pe_b, l1_pn_W, l1_pn_b,
           l1_gru_Wih, l1_gru_Whh, l1_gru_bih, l1_gru_bhh,
           pred_W, pred_b):
    f32 = jnp.float32
    src = jnp.full((E_PAD,), DUMMY, jnp.int32).at[:E].set(
        edge_index[0].astype(jnp.int32))
    dst = jnp.full((E_PAD,), DUMMY, jnp.int32).at[:E].set(
        edge_index[1].astype(jnp.int32))
    hv_pad = jnp.zeros((V_PAD, D), f32).at[:V].set(node_feats)
    ef_pad = jnp.zeros((E_PAD, DE), f32).at[:E].set(edge_feats)

    def col_mat(*cols):
        w = jnp.zeros((G, G), f32)
        for k, c in enumerate(cols):
            w = w.at[:, k].set(c)
        return w

    hv_new, hv_p1_pad, sn_mat = _node_a(
        hv_pad, gc_pn_W.T, gc_pn_b[None], gc_pe1_W[:, :D].T,
        col_mat(gc_pe2_W[0, :G]))
    sn_pad = sn_mat[:, 0]
    efp_pad = _efp_k(ef_pad, gc_pe1_W[:, D:].T, gc_pe1_b[None])

    he1_pad, sdn = _g1_kernel(src, dst, hv_p1_pad, sn_pad, efp_pad)

    he1_3d = he1_pad.reshape(ER, 128, G)
    lg2, bmax = _elogit(he1_3d, sdn.reshape(ER, 128),
                        gc_pe2_W[0:1, G:], jnp.full((1, 128), gc_pe2_b[0]))
    M = jnp.max(bmax)
    e2, she1 = _escale(lg2, jnp.full((1, 128), M), he1_3d)
    e16 = jnp.broadcast_to(e2.reshape(E_PAD)[:, None], (E_PAD, 16))

    agg2c = _scatseq_kernel(dst, she1.reshape(E_PAD, G))
    s2c = _sscat_kernel(dst, e16)
    s16sum = (s2c[0] + s2c[1])[:, :16]

    node, scol, hvproj, bmax = _comb_gc(
        agg2c[0], agg2c[1], s16sum, hv_new, gc_et_W.T, gc_et_b[None],
        gc_gru_Wih.T, gc_gru_Whh.T, gc_gru_bih[None], gc_gru_bhh[None],
        col_mat(l0_pe_W[0, :G], l0_pe_W[0, G:]), l0_pn_W.T, l0_pn_b[None])

    layer_w = (
        (l0_pe_b, l0_gru_Wih, l0_gru_Whh, l0_gru_bih, l0_gru_bhh,
         col_mat(l1_pe_W[0, :G], l1_pe_W[0, G:]), l1_pn_W.T, l1_pn_b[None]),
        (l1_pe_b, l1_gru_Wih, l1_gru_Whh, l1_gru_bih, l1_gru_bhh,
         col_mat(pred_W[0]), jnp.zeros((G, G), f32), jnp.zeros((1, G), f32)),
    )
    for (pe_b, Wih, Whh, bih, bhh, next_packw, next_pnwt, next_pnb) in layer_w:
        b = pe_b[0]
        Mub = _leaky(jnp.max(bmax[:, 0, 0]) + jnp.max(bmax[:, 0, 1]) + b)
        dsc_pad = scol[:, 0]
        ssc_pad = scol[:, 1]

        d_e, s_e = _gat2_kernel(src, dst, dsc_pad, ssc_pad)
        e2 = _elayer(d_e.reshape(ER, 128), s_e.reshape(ER, 128),
                     jnp.full((1, 128), b), jnp.full((1, 128), Mub))
        e16 = jnp.broadcast_to(e2.reshape(E_PAD)[:, None], (E_PAD, 16))

        agg2c = _scat_kernel(src, dst, hvproj, e16)
        s2c = _sscat_kernel(dst, e16)
        s16sum = (s2c[0] + s2c[1])[:, :16]
        node, scol, hvproj, bmax = _comb_layer(
            agg2c[0], agg2c[1], s16sum, node,
            Wih.T, Whh.T, bih[None], bhh[None],
            next_packw, next_pnwt, next_pnb)

    return scol[:V, 0:1] + pred_b


# speedup vs baseline: 5.6006x; 1.0362x over previous
import functools

import jax
import jax.numpy as jnp
from jax import lax
from jax.experimental import pallas as pl
from jax.experimental.pallas import tpu as pltpu
from jax.experimental.pallas import tpu_sc as plsc

V = 10000
E = 320000
D = 128
DE = 16
G = 128

NC = 2
NS = 16
NW = NC * NS
V_PAD = 10240
E_PAD = 327680
EPW = E_PAD // NW
CH = 128
NCHUNK = EPW // CH
RPT = V_PAD // NS
DUMMY = V_PAD - 1

_mesh = plsc.VectorSubcoreMesh(core_axis_name="c", subcore_axis_name="s")


def _leaky_v(x):
    return jnp.where(x > 0, x, x * 0.01)


def _wid():
    return lax.axis_index("s") * NC + lax.axis_index("c")


@functools.partial(
    pl.kernel,
    out_type=(
        jax.ShapeDtypeStruct((E_PAD, G), jnp.float32),
        jax.ShapeDtypeStruct((E_PAD,), jnp.float32),
    ),
    mesh=_mesh,
    scratch_types=[
        [pltpu.VMEM((CH,), jnp.int32)] * 2,
        [pltpu.VMEM((CH,), jnp.int32)] * 2,
        [pltpu.VMEM((CH, G), jnp.float32)] * 2,
        [pltpu.VMEM((CH, G), jnp.float32)] * 2,
        [pltpu.VMEM((CH,), jnp.float32)] * 2,
        [pltpu.SemaphoreType.DMA] * 2,
    ],
)
def _g1_kernel(src_hbm, dst_hbm, hvp1_hbm, sn_hbm, efp_hbm,
               he1_hbm, sdn_hbm,
               idx_s, idx_d, rows, efb, dnv, sem):
    wid = _wid()

    def fire(ch, buf):
        base = wid * EPW + ch * CH
        pltpu.sync_copy(src_hbm.at[pl.ds(base, CH)], idx_s[buf])
        pltpu.sync_copy(dst_hbm.at[pl.ds(base, CH)], idx_d[buf])
        pltpu.async_copy(efp_hbm.at[pl.ds(base, CH)], efb[buf], sem[buf])
        pltpu.async_copy(hvp1_hbm.at[idx_s[buf]], rows[buf], sem[buf])
        pltpu.async_copy(sn_hbm.at[idx_d[buf]], dnv[buf], sem[buf])

    def finish(ch, buf):
        base = wid * EPW + ch * CH
        pltpu.make_async_copy(efp_hbm.at[pl.ds(base, CH)], efb[buf],
                              sem[buf]).wait()
        pltpu.make_async_copy(hvp1_hbm.at[idx_s[buf]], rows[buf],
                              sem[buf]).wait()
        pltpu.make_async_copy(sn_hbm.at[idx_d[buf]], dnv[buf],
                              sem[buf]).wait()
        r = rows[buf]
        ef = efb[buf]

        def edge(g, _):
            for il in range(4):
                i = g * 4 + il
                for j in range(G // 16):
                    u = r[i, pl.ds(j * 16, 16)] + ef[i, pl.ds(j * 16, 16)]
                    r[i, pl.ds(j * 16, 16)] = _leaky_v(u)
            return 0

        lax.fori_loop(0, CH // 4, edge, 0)
        pltpu.sync_copy(r, he1_hbm.at[pl.ds(base, CH)])
        pltpu.sync_copy(dnv[buf], sdn_hbm.at[pl.ds(base, CH)])

    fire(0, 0)

    def body(k, _):
        fire(2 * k + 1, 1)
        finish(2 * k, 0)

        @pl.when(k < NCHUNK // 2 - 1)
        def _():
            fire(2 * k + 2, 0)

        finish(2 * k + 1, 1)
        return 0

    lax.fori_loop(0, NCHUNK // 2, body, 0)


@functools.partial(
    pl.kernel,
    out_type=(
        jax.ShapeDtypeStruct((E_PAD,), jnp.float32),
        jax.ShapeDtypeStruct((E_PAD,), jnp.float32),
    ),
    mesh=_mesh,
    scratch_types=[
        [pltpu.VMEM((CH,), jnp.int32)] * 2,
        [pltpu.VMEM((CH,), jnp.int32)] * 2,
        [pltpu.VMEM((CH,), jnp.float32)] * 2,
        [pltpu.VMEM((CH,), jnp.float32)] * 2,
        [pltpu.SemaphoreType.DMA] * 2,
    ],
)
def _gat2_kernel(src_hbm, dst_hbm, dsc_hbm, ssc_hbm,
                 de_hbm, se_hbm,
                 idx_s, idx_d, dval, sval, sem):
    wid = _wid()

    def fire(ch, buf):
        base = wid * EPW + ch * CH
        pltpu.sync_copy(src_hbm.at[pl.ds(base, CH)], idx_s[buf])
        pltpu.sync_copy(dst_hbm.at[pl.ds(base, CH)], idx_d[buf])
        pltpu.async_copy(dsc_hbm.at[idx_d[buf]], dval[buf], sem[buf])
        pltpu.async_copy(ssc_hbm.at[idx_s[buf]], sval[buf], sem[buf])

    def finish(ch, buf):
        base = wid * EPW + ch * CH
        pltpu.make_async_copy(dsc_hbm.at[idx_d[buf]], dval[buf],
                              sem[buf]).wait()
        pltpu.make_async_copy(ssc_hbm.at[idx_s[buf]], sval[buf],
                              sem[buf]).wait()
        pltpu.sync_copy(dval[buf], de_hbm.at[pl.ds(base, CH)])
        pltpu.sync_copy(sval[buf], se_hbm.at[pl.ds(base, CH)])

    fire(0, 0)

    def body(k, _):
        fire(2 * k + 1, 1)
        finish(2 * k, 0)
        fire(2 * k + 2, 0)
        finish(2 * k + 1, 1)
        return 0

    lax.fori_loop(0, NCHUNK // 2 - 1, body, 0)
    fire(NCHUNK - 1, 1)
    finish(NCHUNK - 2, 0)
    finish(NCHUNK - 1, 1)


@functools.partial(
    pl.kernel,
    out_type=jax.ShapeDtypeStruct((NC, V_PAD, G), jnp.float32),
    mesh=_mesh,
    scratch_types=[
        [pltpu.VMEM((CH,), jnp.int32)] * 1,
        [pltpu.VMEM((1, CH), jnp.int32)] * 1,
        [pltpu.VMEM((CH, G), jnp.float32)] * 1,
        [pltpu.VMEM((CH, 16), jnp.float32)] * 1,
        pltpu.VMEM((16, G), jnp.float32),
        pltpu.VMEM_SHARED((V_PAD, G), jnp.float32),
        [pltpu.SemaphoreType.DMA] * 2,
    ],
)
def _scat_kernel(rid_hbm, dst_hbm, tab_hbm, e16_hbm,
                 agg_out,
                 idx_s, idx_d, rows, e16b, zrow, agg_acc, sem):
    cid = lax.axis_index("c")
    sid = lax.axis_index("s")
    wid = sid * NC + cid

    zf = jnp.zeros((16,), jnp.float32)
    for i in range(16):
        for j in range(G // 16):
            zrow[i, pl.ds(j * 16, 16)] = zf
    base_r = sid * RPT

    def zb(t, _):
        pltpu.sync_copy(zrow, agg_acc.at[pl.ds(base_r + t * 16, 16)])
        return 0

    lax.fori_loop(0, RPT // 16, zb, 0)
    plsc.subcore_barrier()

    def chunk(ch, _):
        base = wid * EPW + ch * CH
        pltpu.async_copy(rid_hbm.at[pl.ds(base, CH)], idx_s[0], sem[0])
        pltpu.async_copy(e16_hbm.at[pl.ds(base, CH)], e16b[0], sem[0])

        @pl.when(ch > 0)
        def _():
            pltpu.make_async_copy(rows[0], agg_acc.at[idx_d[0].at[0]],
                                  sem[1]).wait()

        pltpu.async_copy(dst_hbm.at[pl.ds(base, CH)], idx_d[0].at[0], sem[0])
        pltpu.make_async_copy(rid_hbm.at[pl.ds(base, CH)], idx_s[0],
                              sem[0]).wait()
        pltpu.make_async_copy(dst_hbm.at[pl.ds(base, CH)], idx_d[0].at[0],
                              sem[0]).wait()
        pltpu.make_async_copy(e16_hbm.at[pl.ds(base, CH)], e16b[0],
                              sem[0]).wait()
        pltpu.async_copy(tab_hbm.at[idx_s[0]], rows[0], sem[0]).wait()
        r = rows[0]
        ev = e16b[0]

        def edge(g, _):
            for il in range(4):
                i = g * 4 + il
                eb = ev[i]
                for j in range(G // 16):
                    r[i, pl.ds(j * 16, 16)] = r[i, pl.ds(j * 16, 16)] * eb
            return 0

        lax.fori_loop(0, CH // 4, edge, 0)
        pltpu.async_copy(r, agg_acc.at[idx_d[0].at[0]], sem[1], add=True)
        return 0

    lax.fori_loop(0, NCHUNK, chunk, 0)
    pltpu.make_async_copy(rows[0], agg_acc.at[idx_d[0].at[0]], sem[1]).wait()
    plsc.subcore_barrier()
    pltpu.sync_copy(agg_acc.at[pl.ds(base_r, RPT)],
                    agg_out.at[cid, pl.ds(base_r, RPT)])


@functools.partial(
    pl.kernel,
    out_type=jax.ShapeDtypeStruct((NC, V_PAD, G), jnp.float32),
    mesh=_mesh,
    scratch_types=[
        [pltpu.VMEM((1, CH), jnp.int32)] * 1,
        [pltpu.VMEM((CH, G), jnp.float32)] * 1,
        [pltpu.VMEM((CH, 16), jnp.float32)] * 1,
        pltpu.VMEM((16, G), jnp.float32),
        pltpu.VMEM_SHARED((V_PAD, G), jnp.float32),
        [pltpu.SemaphoreType.DMA] * 1,
    ],
)
def _scatseq_kernel(dst_hbm, tab_hbm,
                    agg_out,
                    idx_d, rows, e16b, zrow, agg_acc, sem):
    cid = lax.axis_index("c")
    sid = lax.axis_index("s")
    wid = sid * NC + cid

    zf = jnp.zeros((16,), jnp.float32)
    for i in range(16):
        for j in range(G // 16):
            zrow[i, pl.ds(j * 16, 16)] = zf
    base_r = sid * RPT

    def zb(t, _):
        pltpu.sync_copy(zrow, agg_acc.at[pl.ds(base_r + t * 16, 16)])
        return 0

    lax.fori_loop(0, RPT // 16, zb, 0)
    plsc.subcore_barrier()

    def chunk(ch, _):
        base = wid * EPW + ch * CH
        pltpu.async_copy(dst_hbm.at[pl.ds(base, CH)], idx_d[0].at[0], sem[0])
        pltpu.async_copy(tab_hbm.at[pl.ds(base, CH)], rows[0], sem[0])
        pltpu.make_async_copy(dst_hbm.at[pl.ds(base, CH)], idx_d[0].at[0],
                              sem[0]).wait()
        pltpu.make_async_copy(tab_hbm.at[pl.ds(base, CH)], rows[0],
                              sem[0]).wait()
        pltpu.sync_copy(rows[0], agg_acc.at[idx_d[0].at[0]], add=True)
        return 0

    lax.fori_loop(0, NCHUNK, chunk, 0)
    plsc.subcore_barrier()
    pltpu.sync_copy(agg_acc.at[pl.ds(base_r, RPT)],
                    agg_out.at[cid, pl.ds(base_r, RPT)])


@functools.partial(
    pl.kernel,
    out_type=jax.ShapeDtypeStruct((NC, V_PAD, G), jnp.float32),
    mesh=_mesh,
    scratch_types=[
        pltpu.VMEM((1, CH), jnp.int32),
        pltpu.VMEM((CH, 16), jnp.float32),
        pltpu.VMEM((CH, G), jnp.float32),
        pltpu.VMEM((16, G), jnp.float32),
        pltpu.VMEM_SHARED((V_PAD, G), jnp.float32),
        [pltpu.SemaphoreType.DMA] * 2,
    ],
)
def _sscat_kernel(dst_hbm, e16_hbm,
                  s_out,
                  idx_d, e16b, e128, zrow, s_acc, sem):
    cid = lax.axis_index("c")
    sid = lax.axis_index("s")
    wid = sid * NC + cid

    zf = jnp.zeros((16,), jnp.float32)
    for i in range(16):
        for j in range(G // 16):
            zrow[i, pl.ds(j * 16, 16)] = zf
    base_r = sid * RPT

    def zb(t, _):
        pltpu.sync_copy(zrow, s_acc.at[pl.ds(base_r + t * 16, 16)])
        return 0

    lax.fori_loop(0, RPT // 16, zb, 0)
    plsc.subcore_barrier()

    def chunk(ch, _):
        base = wid * EPW + ch * CH
        pltpu.async_copy(e16_hbm.at[pl.ds(base, CH)], e16b, sem[0])

        @pl.when(ch > 0)
        def _():
            pltpu.make_async_copy(e128, s_acc.at[idx_d.at[0]],
                                  sem[1]).wait()

        pltpu.async_copy(dst_hbm.at[pl.ds(base, CH)], idx_d.at[0], sem[0])
        pltpu.make_async_copy(dst_hbm.at[pl.ds(base, CH)], idx_d.at[0],
                              sem[0]).wait()
        pltpu.make_async_copy(e16_hbm.at[pl.ds(base, CH)], e16b,
                              sem[0]).wait()

        def edge(g, _):
            for il in range(4):
                i = g * 4 + il
                eb = e16b[i]
                for j in range(G // 16):
                    e128[i, pl.ds(j * 16, 16)] = eb
            return 0

        lax.fori_loop(0, CH // 4, edge, 0)
        pltpu.async_copy(e128, s_acc.at[idx_d.at[0]], sem[1], add=True)
        return 0

    lax.fori_loop(0, NCHUNK, chunk, 0)
    pltpu.make_async_copy(e128, s_acc.at[idx_d.at[0]], sem[1]).wait()
    plsc.subcore_barrier()
    pltpu.sync_copy(s_acc.at[pl.ds(base_r, RPT)],
                    s_out.at[cid, pl.ds(base_r, RPT)])


BRV = 512
NBV = V_PAD // BRV
BRE = 32
ER = E_PAD // 128
NBE = ER // BRE


def _full(shape):
    return pl.BlockSpec(shape, lambda i: tuple(0 for _ in shape))


def _rows(bs, *rest):
    return pl.BlockSpec((bs,) + rest, lambda i: (i,) + tuple(0 for _ in rest))


def _node_a_body(hv_ref, pnwt_ref, pnb_ref, w1nt_ref, w2c_ref,
                 hvnew_ref, hvp1_ref, sn_ref):
    x = hv_ref[...]
    hn = x @ pnwt_ref[...] + pnb_ref[...]
    hn = jnp.where(hn > 0, hn, 0.01 * hn)
    hvnew_ref[...] = hn
    hvp1_ref[...] = x @ w1nt_ref[...]
    sn_ref[...] = hn @ w2c_ref[...]


_node_a = pl.pallas_call(
    _node_a_body,
    grid=(NBV,),
    in_specs=[_rows(BRV, D), _full((D, G)), _full((1, G)), _full((D, G)),
              _full((G, G))],
    out_specs=(_rows(BRV, G), _rows(BRV, G), _rows(BRV, G)),
    out_shape=(jax.ShapeDtypeStruct((V_PAD, G), jnp.float32),
               jax.ShapeDtypeStruct((V_PAD, G), jnp.float32),
               jax.ShapeDtypeStruct((V_PAD, G), jnp.float32)),
)


def _efp_body(ef_ref, w_ref, b_ref, out_ref):
    out_ref[...] = ef_ref[...] @ w_ref[...] + b_ref[...]


_efp_k = pl.pallas_call(
    _efp_body,
    grid=(NBE,),
    in_specs=[_rows(BRE * 128, DE), _full((DE, G)), _full((1, G))],
    out_specs=_rows(BRE * 128, G),
    out_shape=jax.ShapeDtypeStruct((E_PAD, G), jnp.float32),
)


def _elogit_body(he1_ref, sdn_ref, w2e_ref, b2_ref, lg_ref, bmax_ref):
    t = jnp.sum(he1_ref[...] * w2e_ref[...][None], axis=2)
    x = sdn_ref[...] + t + b2_ref[...]
    lg = jnp.where(x > 0, x, 0.01 * x)
    lg_ref[...] = lg
    bmax_ref[...] = jnp.max(lg, axis=0, keepdims=True)[None]


_elogit = pl.pallas_call(
    _elogit_body,
    grid=(NBE,),
    in_specs=[_rows(BRE, 128, G), _rows(BRE, 128), _full((1, G)),
              _full((1, 128))],
    out_specs=(_rows(BRE, 128), _rows(1, 1, 128)),
    out_shape=(jax.ShapeDtypeStruct((ER, 128), jnp.float32),
               jax.ShapeDtypeStruct((NBE, 1, 128), jnp.float32)),
)


def _escale_body(lg_ref, m_ref, he1_ref, e_ref, sc_ref):
    e = jnp.exp(lg_ref[...] - m_ref[...])
    e_ref[...] = e
    sc_ref[...] = he1_ref[...] * e[:, :, None]


_escale = pl.pallas_call(
    _escale_body,
    grid=(NBE,),
    in_specs=[_rows(BRE, 128), _full((1, 128)), _rows(BRE, 128, G)],
    out_specs=(_rows(BRE, 128), _rows(BRE, 128, G)),
    out_shape=(jax.ShapeDtypeStruct((ER, 128), jnp.float32),
               jax.ShapeDtypeStruct((ER, 128, G), jnp.float32)),
)


def _elayer_body(d_ref, s_ref, b_ref, m_ref, e_ref):
    x = d_ref[...] + s_ref[...] + b_ref[...]
    lg = jnp.where(x > 0, x, 0.01 * x)
    e_ref[...] = jnp.exp(lg - m_ref[...])


_elayer = pl.pallas_call(
    _elayer_body,
    grid=(NBE,),
    in_specs=[_rows(BRE, 128), _rows(BRE, 128), _full((1, 128)),
              _full((1, 128))],
    out_specs=_rows(BRE, 128),
    out_shape=jax.ShapeDtypeStruct((ER, 128), jnp.float32),
)


def _gru_block(ctx, h, wih_t, whh_t, bih, bhh):
    gi = ctx @ wih_t + bih
    gh = h @ whh_t + bhh
    r = jax.nn.sigmoid(gi[:, :G] + gh[:, :G])
    z = jax.nn.sigmoid(gi[:, G:2 * G] + gh[:, G:2 * G])
    n = jnp.tanh(gi[:, 2 * G:] + r * gh[:, 2 * G:])
    node = (1.0 - z) * n + z * h
    return jnp.maximum(node, 0.0)


def _tables_block(node, packw, pnwt, pnb, scol_ref, hvproj_ref, bmax_ref):
    scol = node @ packw
    scol_ref[...] = scol
    hvproj_ref[...] = node @ pnwt + pnb
    bmax_ref[...] = jnp.max(scol, axis=0, keepdims=True)[None]


def _comb_gc_body(agg0_ref, agg1_ref, s16_ref, hvnew_ref, etwt_ref, etb_ref,
                  wih_ref, whh_ref, bih_ref, bhh_ref,
                  packw_ref, pnwt_ref, pnb_ref,
                  node_ref, scol_ref, hvproj_ref, bmax_ref):
    agg = agg0_ref[...] + agg1_ref[...]
    s = s16_ref[...][:, 0:1]
    denom = s + 1e-9
    c = (agg @ etwt_ref[...]) / denom + (s / denom) * etb_ref[...]
    ctx = jnp.where(c > 0, c, jnp.exp(c) - 1.0)
    node = _gru_block(ctx, hvnew_ref[...], wih_ref[...], whh_ref[...],
                      bih_ref[...], bhh_ref[...])
    node_ref[...] = node
    _tables_block(node, packw_ref[...], pnwt_ref[...], pnb_ref[...],
                  scol_ref, hvproj_ref, bmax_ref)


_comb_gc = pl.pallas_call(
    _comb_gc_body,
    grid=(NBV,),
    in_specs=[_rows(BRV, G), _rows(BRV, G), _rows(BRV, 16), _rows(BRV, G),
              _full((G, G)), _full((1, G)),
              _full((G, 3 * G)), _full((G, 3 * G)), _full((1, 3 * G)),
              _full((1, 3 * G)),
              _full((G, G)), _full((G, G)), _full((1, G))],
    out_specs=(_rows(BRV, G), _rows(BRV, G), _rows(BRV, G),
               _rows(1, 1, 128)),
    out_shape=(jax.ShapeDtypeStruct((V_PAD, G), jnp.float32),
               jax.ShapeDtypeStruct((V_PAD, G), jnp.float32),
               jax.ShapeDtypeStruct((V_PAD, G), jnp.float32),
               jax.ShapeDtypeStruct((NBV, 1, 128), jnp.float32)),
)


def _comb_layer_body(agg0_ref, agg1_ref, s16_ref, h_ref,
                     wih_ref, whh_ref, bih_ref, bhh_ref,
                     packw_ref, pnwt_ref, pnb_ref,
                     node_ref, scol_ref, hvproj_ref, bmax_ref):
    agg = agg0_ref[...] + agg1_ref[...]
    s = s16_ref[...][:, 0:1]
    c = agg / (s + 1e-9)
    ctx = jnp.where(c > 0, c, jnp.exp(c) - 1.0)
    node = _gru_block(ctx, h_ref[...], wih_ref[...], whh_ref[...],
                      bih_ref[...], bhh_ref[...])
    node_ref[...] = node
    _tables_block(node, packw_ref[...], pnwt_ref[...], pnb_ref[...],
                  scol_ref, hvproj_ref, bmax_ref)


_comb_layer = pl.pallas_call(
    _comb_layer_body,
    grid=(NBV,),
    in_specs=[_rows(BRV, G), _rows(BRV, G), _rows(BRV, 16), _rows(BRV, G),
              _full((G, 3 * G)), _full((G, 3 * G)), _full((1, 3 * G)),
              _full((1, 3 * G)),
              _full((G, G)), _full((G, G)), _full((1, G))],
    out_specs=(_rows(BRV, G), _rows(BRV, G), _rows(BRV, G),
               _rows(1, 1, 128)),
    out_shape=(jax.ShapeDtypeStruct((V_PAD, G), jnp.float32),
               jax.ShapeDtypeStruct((V_PAD, G), jnp.float32),
               jax.ShapeDtypeStruct((V_PAD, G), jnp.float32),
               jax.ShapeDtypeStruct((NBV, 1, 128), jnp.float32)),
)


def _leaky(x):
    return jax.nn.leaky_relu(x, negative_slope=0.01)


def _gru_update(x, h, W_ih, W_hh, b_ih, b_hh):
    gi = x @ W_ih.T + b_ih
    gh = h @ W_hh.T + b_hh
    i_r, i_z, i_n = jnp.split(gi, 3, axis=1)
    h_r, h_z, h_n = jnp.split(gh, 3, axis=1)
    r = jax.nn.sigmoid(i_r + h_r)
    z = jax.nn.sigmoid(i_z + h_z)
    n = jnp.tanh(i_n + r * h_n)
    return (1.0 - z) * n + z * h


def kernel(node_feats, edge_feats, edge_index,
           gc_pn_W, gc_pn_b, gc_pe1_W, gc_pe1_b, gc_pe2_W, gc_pe2_b,
           gc_et_W, gc_et_b, gc_gru_Wih, gc_gru_Whh, gc_gru_bih, gc_gru_bhh,
           l0_pe_W, l0_pe_b, l0_pn_W, l0_pn_b,
           l0_gru_Wih, l0_gru_Whh, l0_gru_bih, l0_gru_bhh,
           l1_pe_W, l1_pe_b, l1_pn_W, l1_pn_b,
           l1_gru_Wih, l1_gru_Whh, l1_gru_bih, l1_gru_bhh,
           pred_W, pred_b):
    f32 = jnp.float32
    src = jnp.full((E_PAD,), DUMMY, jnp.int32).at[:E].set(
        edge_index[0].astype(jnp.int32))
    dst = jnp.full((E_PAD,), DUMMY, jnp.int32).at[:E].set(
        edge_index[1].astype(jnp.int32))
    hv_pad = jnp.zeros((V_PAD, D), f32).at[:V].set(node_feats)
    ef_pad = jnp.zeros((E_PAD, DE), f32).at[:E].set(edge_feats)

    def col_mat(*cols):
        w = jnp.zeros((G, G), f32)
        for k, c in enumerate(cols):
            w = w.at[:, k].set(c)
        return w

    hv_new, hv_p1_pad, sn_mat = _node_a(
        hv_pad, gc_pn_W.T, gc_pn_b[None], gc_pe1_W[:, :D].T,
        col_mat(gc_pe2_W[0, :G]))
    sn_pad = sn_mat[:, 0]
    efp_pad = _efp_k(ef_pad, gc_pe1_W[:, D:].T, gc_pe1_b[None])

    he1_pad, sdn = _g1_kernel(src, dst, hv_p1_pad, sn_pad, efp_pad)

    he1_3d = he1_pad.reshape(ER, 128, G)
    lg2, bmax = _elogit(he1_3d, sdn.reshape(ER, 128),
                        gc_pe2_W[0:1, G:], jnp.full((1, 128), gc_pe2_b[0]))
    M = jnp.max(bmax)
    e2, she1 = _escale(lg2, jnp.full((1, 128), M), he1_3d)
    e16 = jnp.broadcast_to(e2.reshape(E_PAD)[:, None], (E_PAD, 16))

    agg2c = _scatseq_kernel(dst, she1.reshape(E_PAD, G))
    s2c = _sscat_kernel(dst, e16)
    s16sum = (s2c[0] + s2c[1])[:, :16]

    node, scol, hvproj, bmax = _comb_gc(
        agg2c[0], agg2c[1], s16sum, hv_new, gc_et_W.T, gc_et_b[None],
        gc_gru_Wih.T, gc_gru_Whh.T, gc_gru_bih[None], gc_gru_bhh[None],
        col_mat(l0_pe_W[0, :G], l0_pe_W[0, G:]), l0_pn_W.T, l0_pn_b[None])

    layer_w = (
        (l0_pe_b, l0_gru_Wih, l0_gru_Whh, l0_gru_bih, l0_gru_bhh,
         col_mat(l1_pe_W[0, :G], l1_pe_W[0, G:]), l1_pn_W.T, l1_pn_b[None]),
        (l1_pe_b, l1_gru_Wih, l1_gru_Whh, l1_gru_bih, l1_gru_bhh,
         col_mat(pred_W[0]), jnp.zeros((G, G), f32), jnp.zeros((1, G), f32)),
    )
    for (pe_b, Wih, Whh, bih, bhh, next_packw, next_pnwt, next_pnb) in layer_w:
        b = pe_b[0]
        Mub = _leaky(jnp.max(bmax[:, 0, 0]) + jnp.max(bmax[:, 0, 1]) + b)
        dsc_pad = scol[:, 0]
        ssc_pad = scol[:, 1]

        d_e, s_e = _gat2_kernel(src, dst, dsc_pad, ssc_pad)
        e2 = _elayer(d_e.reshape(ER, 128), s_e.reshape(ER, 128),
                     jnp.full((1, 128), b), jnp.full((1, 128), Mub))
        e16 = jnp.broadcast_to(e2.reshape(E_PAD)[:, None], (E_PAD, 16))

        agg2c = _scat_kernel(src, dst, hvproj, e16)
        s2c = _sscat_kernel(dst, e16)
        s16sum = (s2c[0] + s2c[1])[:, :16]
        node, scol, hvproj, bmax = _comb_layer(
            agg2c[0], agg2c[1], s16sum, node,
            Wih.T, Whh.T, bih[None], bhh[None],
            next_packw, next_pnwt, next_pnb)

    return scol[:V, 0:1] + pred_b
